# Initial kernel scaffold; baseline (speedup 1.0000x reference)
import jax
import jax.numpy as jnp
from jax.experimental import pallas as pl


def kernel(x, edge_index, pair, W1, b1, W2, b2, Wm1, bm1, Wm2, bm2):
    raise NotImplementedError("write your pallas kernel here")


# speedup vs baseline: 11.9264x; 11.9264x over previous
import functools

import jax
import jax.numpy as jnp
from jax import lax
from jax.experimental import pallas as pl
from jax.experimental.pallas import tpu as pltpu
from jax.experimental.pallas import tpu_sc as plsc

N = 10000
D = 128
E = 320000
NPAIR = 65536
NC, NS, NW = 2, 16, 32
D2 = D // 2
EPT = E // NW
EPS = E // NS
KE = 80
NCE = EPT // KE
NCP2 = EPS // KE
CR = 400
CH = N // CR
CK = (CH + NS - 1) // NS + 1
KP = 128
PPT = NPAIR // NW
NCP = PPT // KP


def _mesh():
    return plsc.VectorSubcoreMesh(
        core_axis_name="c", subcore_axis_name="s",
        num_cores=NC, num_subcores=NS)


@functools.partial(
    pl.kernel,
    out_type=jax.ShapeDtypeStruct((NC, N, 16), jnp.float32),
    mesh=_mesh(),
    scratch_types=[
        pltpu.VMEM((NCE, KE), jnp.int32),
        pltpu.VMEM((KE,), jnp.int32),
        pltpu.VMEM((KE, 16), jnp.float32),
        pltpu.VMEM((CR, 16), jnp.float32),
        pltpu.VMEM_SHARED((N, 16), jnp.float32),
    ],
    compiler_params=pltpu.CompilerParams(use_tc_tiling_on_sc=False),
)
def _sc_degree(dst_hbm, degp_hbm, idxbuf, idxchunk, ones, stage, shacc):
    c = lax.axis_index("c")
    s = lax.axis_index("s")
    wid = c * NS + s
    pltpu.sync_copy(dst_hbm.at[wid], idxbuf)

    one16 = jnp.ones((16,), jnp.float32)
    zero16 = jnp.zeros((16,), jnp.float32)

    def fill_ones(i, carry):
        ones[i, :] = one16
        return carry
    lax.fori_loop(0, KE, fill_ones, 0)

    def fill_zero(i, carry):
        stage[i, :] = zero16
        return carry
    lax.fori_loop(0, CR, fill_zero, 0)
    for k in range(CK):
        m = s + k * NS

        @pl.when(m < CH)
        def _():
            pltpu.sync_copy(stage, shacc.at[pl.ds(m * CR, CR), :])
    plsc.subcore_barrier()

    def edge_body(j, carry):
        for u in range(KE // 16):
            idxchunk[pl.ds(u * 16, 16)] = idxbuf[j, pl.ds(u * 16, 16)]
        pltpu.sync_copy(ones, shacc.at[idxchunk], add=True)
        return carry
    lax.fori_loop(0, NCE, edge_body, 0)
    plsc.subcore_barrier()

    for k in range(CK):
        m = s + k * NS

        @pl.when(m < CH)
        def _():
            pltpu.sync_copy(shacc.at[pl.ds(m * CR, CR), :], stage)
            pltpu.sync_copy(stage, degp_hbm.at[c, pl.ds(m * CR, CR), :])


@functools.partial(
    pl.kernel,
    out_type=jax.ShapeDtypeStruct((NC, N, D2), jnp.float32),
    mesh=_mesh(),
    scratch_types=[
        pltpu.VMEM((NCP2, KE), jnp.int32),
        pltpu.VMEM((NCP2, KE), jnp.int32),
        pltpu.VMEM((KE,), jnp.int32),
        pltpu.VMEM((KE, D2), jnp.float32),
        pltpu.VMEM((CR, D2), jnp.float32),
        pltpu.SemaphoreType.DMA,
        pltpu.VMEM_SHARED((N, D2), jnp.float32),
    ],
    compiler_params=pltpu.CompilerParams(use_tc_tiling_on_sc=False),
)
def _sc_prop(ga_hbm, gb_hbm, src_hbm, dst_hbm, pp_hbm,
             srcbuf, dstbuf, dstchunk, rows, zbuf, sem, shacc):
    c = lax.axis_index("c")
    s = lax.axis_index("s")
    pltpu.sync_copy(src_hbm.at[s], srcbuf)
    pltpu.sync_copy(dst_hbm.at[s], dstbuf)

    zero16 = jnp.zeros((16,), jnp.float32)

    def zb(i, carry):
        for u in range(D2 // 16):
            zbuf[i, pl.ds(u * 16, 16)] = zero16
        return carry
    lax.fori_loop(0, CR, zb, 0)
    for k in range(CK):
        m = s + k * NS

        @pl.when(m < CH)
        def _():
            pltpu.sync_copy(zbuf, shacc.at[pl.ds(m * CR, CR), :])
    plsc.subcore_barrier()

    def edge_body(j, carry):
        for u in range(KE // 16):
            dstchunk[pl.ds(u * 16, 16)] = dstbuf[j, pl.ds(u * 16, 16)]

        @pl.when(c == 0)
        def _():
            pltpu.async_copy(ga_hbm.at[srcbuf.at[j]], rows, sem).wait()

        @pl.when(c == 1)
        def _():
            pltpu.async_copy(gb_hbm.at[srcbuf.at[j]], rows, sem).wait()
        pltpu.sync_copy(rows, shacc.at[dstchunk], add=True)
        return carry
    lax.fori_loop(0, NCP2, edge_body, 0)
    plsc.subcore_barrier()

    for k in range(CK):
        m = s + k * NS

        @pl.when(m < CH)
        def _():
            pltpu.sync_copy(shacc.at[pl.ds(m * CR, CR), :], zbuf)
            pltpu.sync_copy(zbuf, pp_hbm.at[c, pl.ds(m * CR, CR), :])


@functools.partial(
    pl.kernel,
    out_type=jax.ShapeDtypeStruct((NPAIR, D), jnp.float32),
    mesh=_mesh(),
    scratch_types=[
        pltpu.VMEM((NCP, KP), jnp.int32),
        pltpu.VMEM((NCP, KP), jnp.int32),
        pltpu.VMEM((KP, D), jnp.float32),
        pltpu.VMEM((KP, D), jnp.float32),
        pltpu.VMEM((KP,), jnp.int32),
        pltpu.SemaphoreType.DMA,
        pltpu.VMEM_SHARED((NS * KP, D), jnp.float32),
    ],
    compiler_params=pltpu.CompilerParams(use_tc_tiling_on_sc=False),
)
def _sc_pair(p_hbm, q_hbm, i0_hbm, i1_hbm, ape_hbm,
             i0buf, i1buf, bufa, bufb, idbuf, sem, shp):
    c = lax.axis_index("c")
    s = lax.axis_index("s")
    wid = c * NS + s
    pltpu.sync_copy(i0_hbm.at[wid], i0buf)
    pltpu.sync_copy(i1_hbm.at[wid], i1buf)

    iota16 = lax.iota(jnp.int32, 16)
    for u in range(KP // 16):
        idbuf[pl.ds(u * 16, 16)] = s * KP + u * 16 + iota16

    base = wid * PPT

    def pbody(j, carry):
        pltpu.async_copy(p_hbm.at[i0buf.at[j]], bufa, sem).wait()
        pltpu.async_copy(q_hbm.at[i1buf.at[j]], bufb, sem).wait()
        pltpu.sync_copy(bufa, shp.at[pl.ds(s * KP, KP), :])
        pltpu.sync_copy(bufb, shp.at[idbuf], add=True)
        pltpu.sync_copy(shp.at[pl.ds(s * KP, KP), :], bufa)
        pltpu.sync_copy(bufa, ape_hbm.at[pl.ds(base + j * KP, KP), :])
        return carry
    lax.fori_loop(0, NCP, pbody, 0)


_RB = 1000
_GRID = N // _RB


def _dinv(degp_ref):
    deg = 1.0 + degp_ref[0, :, 0] + degp_ref[1, :, 0]
    return lax.rsqrt(deg)[:, None]


def _tc_in_body(x_ref, w1_ref, degp_ref, oa_ref, ob_ref):
    g = jnp.dot(x_ref[...], w1_ref[...],
                preferred_element_type=jnp.float32) * _dinv(degp_ref)
    oa_ref[...] = g[:, :D2]
    ob_ref[...] = g[:, D2:]


def _tc_mid_body(pp_ref, ga_ref, gb_ref, degp_ref, b1_ref, w2_ref,
                 oa_ref, ob_ref):
    dinv = _dinv(degp_ref)
    h = jnp.concatenate(
        [pp_ref[0] + ga_ref[...], pp_ref[1] + gb_ref[...]], axis=-1)
    h = jnp.maximum(h * dinv + b1_ref[...], 0.0)
    g = jnp.dot(h, w2_ref[...], preferred_element_type=jnp.float32) * dinv
    oa_ref[...] = g[:, :D2]
    ob_ref[...] = g[:, D2:]


def _tc_out_body(pp_ref, ga_ref, gb_ref, degp_ref, b2_ref, wm1_ref,
                 p_ref, q_ref):
    dinv = _dinv(degp_ref)
    h2 = jnp.concatenate(
        [pp_ref[0] + ga_ref[...], pp_ref[1] + gb_ref[...]], axis=-1)
    h2 = h2 * dinv + b2_ref[...]
    p_ref[...] = jnp.dot(h2, wm1_ref[:D, :],
                         preferred_element_type=jnp.float32)
    q_ref[...] = jnp.dot(h2, wm1_ref[D:, :],
                         preferred_element_type=jnp.float32)


_PB = 2048


def _tc_head_body(ape_ref, bm1_ref, wm2_ref, bm2_ref, o_ref):
    t = jnp.maximum(ape_ref[...] + bm1_ref[...], 0.0)
    z = jnp.dot(t, wm2_ref[...], preferred_element_type=jnp.float32)
    o_ref[...] = jax.nn.sigmoid(z + bm2_ref[0, 0])


def _row_spec(shape):
    nd = len(shape)
    if nd == 2:
        return pl.BlockSpec((_RB, shape[1]), lambda i: (i, 0))
    return pl.BlockSpec((shape[0], _RB, shape[2]), lambda i: (0, i, 0))


def _full_spec(shape):
    nd = len(shape)
    return pl.BlockSpec(shape, lambda i: (0,) * nd)


def _tc_in(x, w1, degp):
    half = jax.ShapeDtypeStruct((N, D2), jnp.float32)
    return pl.pallas_call(
        _tc_in_body,
        grid=(_GRID,),
        in_specs=[_row_spec(x.shape), _full_spec(w1.shape),
                  _row_spec(degp.shape)],
        out_specs=[_row_spec((N, D2)), _row_spec((N, D2))],
        out_shape=[half, half],
    )(x, w1, degp)


def _tc_mid(pp, ga, gb, degp, b1, w2):
    half = jax.ShapeDtypeStruct((N, D2), jnp.float32)
    return pl.pallas_call(
        _tc_mid_body,
        grid=(_GRID,),
        in_specs=[_row_spec(pp.shape), _row_spec(ga.shape),
                  _row_spec(gb.shape), _row_spec(degp.shape),
                  _full_spec(b1.shape), _full_spec(w2.shape)],
        out_specs=[_row_spec((N, D2)), _row_spec((N, D2))],
        out_shape=[half, half],
    )(pp, ga, gb, degp, b1, w2)


def _tc_out(pp, ga, gb, degp, b2, wm1):
    full = jax.ShapeDtypeStruct((N, D), jnp.float32)
    return pl.pallas_call(
        _tc_out_body,
        grid=(_GRID,),
        in_specs=[_row_spec(pp.shape), _row_spec(ga.shape),
                  _row_spec(gb.shape), _row_spec(degp.shape),
                  _full_spec(b2.shape), _full_spec(wm1.shape)],
        out_specs=[_row_spec((N, D)), _row_spec((N, D))],
        out_shape=[full, full],
    )(pp, ga, gb, degp, b2, wm1)


def _tc_head(ape, bm1, wm2, bm2):
    return pl.pallas_call(
        _tc_head_body,
        grid=(NPAIR // _PB,),
        in_specs=[pl.BlockSpec((_PB, D), lambda i: (i, 0)),
                  _full_spec(bm1.shape), _full_spec(wm2.shape),
                  _full_spec(bm2.shape)],
        out_specs=pl.BlockSpec((_PB, 1), lambda i: (i, 0)),
        out_shape=jax.ShapeDtypeStruct((NPAIR, 1), jnp.float32),
    )(ape, bm1, wm2, bm2)


def kernel(x, edge_index, pair, W1, b1, W2, b2, Wm1, bm1, Wm2, bm2):
    src32 = edge_index[0].astype(jnp.int32)
    dst32 = edge_index[1].astype(jnp.int32)
    dst_deg = dst32.reshape(NW, NCE, KE)
    src_prop = src32.reshape(NS, NCP2, KE)
    dst_prop = dst32.reshape(NS, NCP2, KE)
    p0 = pair[:, 0].astype(jnp.int32).reshape(NW, NCP, KP)
    p1 = pair[:, 1].astype(jnp.int32).reshape(NW, NCP, KP)

    degp = _sc_degree(dst_deg)
    g1a, g1b = _tc_in(x, W1, degp)
    pp1 = _sc_prop(g1a, g1b, src_prop, dst_prop)
    g2a, g2b = _tc_mid(pp1, g1a, g1b, degp, b1.reshape(1, D), W2)
    pp2 = _sc_prop(g2a, g2b, src_prop, dst_prop)
    P, Q = _tc_out(pp2, g2a, g2b, degp, b2.reshape(1, D), Wm1)
    ape = _sc_pair(P, Q, p0, p1)
    return _tc_head(ape, bm1.reshape(1, D), Wm2, bm2.reshape(1, 1))


# speedup vs baseline: 17.0238x; 1.4274x over previous
import functools

import jax
import jax.numpy as jnp
from jax import lax
from jax.experimental import pallas as pl
from jax.experimental.pallas import tpu as pltpu
from jax.experimental.pallas import tpu_sc as plsc

N = 10000
D = 128
E = 320000
NPAIR = 65536
NC, NS, NW = 2, 16, 32
D2 = D // 2
EPT = E // NW
EPS = E // NS
KE = 80
NCE = EPT // KE
NCP2 = EPS // KE
CR = 400
CH = N // CR
CK = (CH + NS - 1) // NS + 1
KP = 128
PPT = NPAIR // NW
NCP = PPT // KP


def _mesh():
    return plsc.VectorSubcoreMesh(
        core_axis_name="c", subcore_axis_name="s",
        num_cores=NC, num_subcores=NS)


@functools.partial(
    pl.kernel,
    out_type=jax.ShapeDtypeStruct((NC, N, 16), jnp.float32),
    mesh=_mesh(),
    scratch_types=[
        pltpu.VMEM((NCE, KE), jnp.int32),
        pltpu.VMEM((KE,), jnp.int32),
        pltpu.VMEM((KE, 16), jnp.float32),
        pltpu.VMEM((CR, 16), jnp.float32),
        pltpu.VMEM_SHARED((N, 16), jnp.float32),
    ],
    compiler_params=pltpu.CompilerParams(use_tc_tiling_on_sc=False),
)
def _sc_degree(dst_hbm, degp_hbm, idxbuf, idxchunk, ones, stage, shacc):
    c = lax.axis_index("c")
    s = lax.axis_index("s")
    wid = c * NS + s
    pltpu.sync_copy(dst_hbm.at[wid], idxbuf)

    one16 = jnp.ones((16,), jnp.float32)
    zero16 = jnp.zeros((16,), jnp.float32)

    def fill_ones(i, carry):
        ones[i, :] = one16
        return carry
    lax.fori_loop(0, KE, fill_ones, 0)

    def fill_zero(i, carry):
        stage[i, :] = zero16
        return carry
    lax.fori_loop(0, CR, fill_zero, 0)
    for k in range(CK):
        m = s + k * NS

        @pl.when(m < CH)
        def _():
            pltpu.sync_copy(stage, shacc.at[pl.ds(m * CR, CR), :])
    plsc.subcore_barrier()

    def edge_body(j, carry):
        for u in range(KE // 16):
            idxchunk[pl.ds(u * 16, 16)] = idxbuf[j, pl.ds(u * 16, 16)]
        pltpu.sync_copy(ones, shacc.at[idxchunk], add=True)
        return carry
    lax.fori_loop(0, NCE, edge_body, 0)
    plsc.subcore_barrier()

    for k in range(CK):
        m = s + k * NS

        @pl.when(m < CH)
        def _():
            pltpu.sync_copy(shacc.at[pl.ds(m * CR, CR), :], stage)
            pltpu.sync_copy(stage, degp_hbm.at[c, pl.ds(m * CR, CR), :])


@functools.partial(
    pl.kernel,
    out_type=jax.ShapeDtypeStruct((NC, N, D2), jnp.float32),
    mesh=_mesh(),
    scratch_types=[
        pltpu.VMEM((NCP2, KE), jnp.int32),
        pltpu.VMEM((NCP2, KE), jnp.int32),
        pltpu.VMEM((KE,), jnp.int32),
        pltpu.VMEM((KE,), jnp.int32),
        pltpu.VMEM((KE, D2), jnp.float32),
        pltpu.VMEM((KE, D2), jnp.float32),
        pltpu.VMEM((CR, D2), jnp.float32),
        pltpu.SemaphoreType.DMA,
        pltpu.SemaphoreType.DMA,
        pltpu.VMEM_SHARED((N, D2), jnp.float32),
    ],
    compiler_params=pltpu.CompilerParams(use_tc_tiling_on_sc=False),
)
def _sc_prop(ga_hbm, gb_hbm, src_hbm, dst_hbm, pp_hbm,
             srcbuf, dstbuf, dchunka, dchunkb, rowsa, rowsb, zbuf,
             sema, semb, shacc):
    c = lax.axis_index("c")
    s = lax.axis_index("s")
    pltpu.sync_copy(src_hbm.at[s], srcbuf)
    pltpu.sync_copy(dst_hbm.at[s], dstbuf)

    zero16 = jnp.zeros((16,), jnp.float32)

    def zb(i, carry):
        for u in range(D2 // 16):
            zbuf[i, pl.ds(u * 16, 16)] = zero16
        return carry
    lax.fori_loop(0, CR, zb, 0)
    for k in range(CK):
        m = s + k * NS

        @pl.when(m < CH)
        def _():
            pltpu.sync_copy(zbuf, shacc.at[pl.ds(m * CR, CR), :])
    plsc.subcore_barrier()

    def copy_dst_idx(j, chunk):
        for u in range(KE // 16):
            chunk[pl.ds(u * 16, 16)] = dstbuf[j, pl.ds(u * 16, 16)]

    def run_edges(gsrc):
        copy_dst_idx(0, dchunka)
        pltpu.async_copy(gsrc.at[srcbuf.at[0]], rowsa, sema)

        def body(t, carry):
            j0 = 2 * t
            j1 = j0 + 1
            copy_dst_idx(j1, dchunkb)
            pltpu.async_copy(gsrc.at[srcbuf.at[j1]], rowsb, semb)
            pltpu.make_async_copy(gsrc.at[srcbuf.at[j0]], rowsa, sema).wait()
            pltpu.sync_copy(rowsa, shacc.at[dchunka], add=True)

            @pl.when(t + 1 < NCP2 // 2)
            def _():
                copy_dst_idx(j0 + 2, dchunka)
                pltpu.async_copy(gsrc.at[srcbuf.at[j0 + 2]], rowsa, sema)
            pltpu.make_async_copy(gsrc.at[srcbuf.at[j1]], rowsb, semb).wait()
            pltpu.sync_copy(rowsb, shacc.at[dchunkb], add=True)
            return carry
        lax.fori_loop(0, NCP2 // 2, body, 0)

    @pl.when(c == 0)
    def _():
        run_edges(ga_hbm)

    @pl.when(c == 1)
    def _():
        run_edges(gb_hbm)
    plsc.subcore_barrier()

    for k in range(CK):
        m = s + k * NS

        @pl.when(m < CH)
        def _():
            pltpu.sync_copy(shacc.at[pl.ds(m * CR, CR), :], zbuf)
            pltpu.sync_copy(zbuf, pp_hbm.at[c, pl.ds(m * CR, CR), :])


@functools.partial(
    pl.kernel,
    out_type=jax.ShapeDtypeStruct((NPAIR, D), jnp.float32),
    mesh=_mesh(),
    scratch_types=[
        pltpu.VMEM((NCP, KP), jnp.int32),
        pltpu.VMEM((NCP, KP), jnp.int32),
        pltpu.VMEM((KP, D), jnp.float32),
        pltpu.VMEM((KP, D), jnp.float32),
        pltpu.VMEM((KP,), jnp.int32),
        pltpu.SemaphoreType.DMA,
        pltpu.VMEM_SHARED((NS * KP, D), jnp.float32),
    ],
    compiler_params=pltpu.CompilerParams(use_tc_tiling_on_sc=False),
)
def _sc_pair(p_hbm, q_hbm, i0_hbm, i1_hbm, ape_hbm,
             i0buf, i1buf, bufa, bufb, idbuf, sem, shp):
    c = lax.axis_index("c")
    s = lax.axis_index("s")
    wid = c * NS + s
    pltpu.sync_copy(i0_hbm.at[wid], i0buf)
    pltpu.sync_copy(i1_hbm.at[wid], i1buf)

    iota16 = lax.iota(jnp.int32, 16)
    for u in range(KP // 16):
        idbuf[pl.ds(u * 16, 16)] = s * KP + u * 16 + iota16

    base = wid * PPT

    def pbody(j, carry):
        pltpu.async_copy(p_hbm.at[i0buf.at[j]], bufa, sem).wait()
        pltpu.async_copy(q_hbm.at[i1buf.at[j]], bufb, sem).wait()
        pltpu.sync_copy(bufa, shp.at[pl.ds(s * KP, KP), :])
        pltpu.sync_copy(bufb, shp.at[idbuf], add=True)
        pltpu.sync_copy(shp.at[pl.ds(s * KP, KP), :], bufa)
        pltpu.sync_copy(bufa, ape_hbm.at[pl.ds(base + j * KP, KP), :])
        return carry
    lax.fori_loop(0, NCP, pbody, 0)


_RB = 1000
_GRID = N // _RB


def _dinv(degp_ref):
    deg = 1.0 + degp_ref[0, :, 0] + degp_ref[1, :, 0]
    return lax.rsqrt(deg)[:, None]


def _tc_in_body(x_ref, w1_ref, degp_ref, oa_ref, ob_ref):
    g = jnp.dot(x_ref[...], w1_ref[...],
                preferred_element_type=jnp.float32) * _dinv(degp_ref)
    oa_ref[...] = g[:, :D2]
    ob_ref[...] = g[:, D2:]


def _tc_mid_body(pp_ref, ga_ref, gb_ref, degp_ref, b1_ref, w2_ref,
                 oa_ref, ob_ref):
    dinv = _dinv(degp_ref)
    h = jnp.concatenate(
        [pp_ref[0] + ga_ref[...], pp_ref[1] + gb_ref[...]], axis=-1)
    h = jnp.maximum(h * dinv + b1_ref[...], 0.0)
    g = jnp.dot(h, w2_ref[...], preferred_element_type=jnp.float32) * dinv
    oa_ref[...] = g[:, :D2]
    ob_ref[...] = g[:, D2:]


def _tc_out_body(pp_ref, ga_ref, gb_ref, degp_ref, b2_ref, wm1_ref,
                 p_ref, q_ref):
    dinv = _dinv(degp_ref)
    h2 = jnp.concatenate(
        [pp_ref[0] + ga_ref[...], pp_ref[1] + gb_ref[...]], axis=-1)
    h2 = h2 * dinv + b2_ref[...]
    p_ref[...] = jnp.dot(h2, wm1_ref[:D, :],
                         preferred_element_type=jnp.float32)
    q_ref[...] = jnp.dot(h2, wm1_ref[D:, :],
                         preferred_element_type=jnp.float32)


_PB = 2048


def _tc_head_body(ape_ref, bm1_ref, wm2_ref, bm2_ref, o_ref):
    t = jnp.maximum(ape_ref[...] + bm1_ref[...], 0.0)
    z = jnp.dot(t, wm2_ref[...], preferred_element_type=jnp.float32)
    o_ref[...] = jax.nn.sigmoid(z + bm2_ref[0, 0])


def _row_spec(shape):
    nd = len(shape)
    if nd == 2:
        return pl.BlockSpec((_RB, shape[1]), lambda i: (i, 0))
    return pl.BlockSpec((shape[0], _RB, shape[2]), lambda i: (0, i, 0))


def _full_spec(shape):
    nd = len(shape)
    return pl.BlockSpec(shape, lambda i: (0,) * nd)


def _tc_in(x, w1, degp):
    half = jax.ShapeDtypeStruct((N, D2), jnp.float32)
    return pl.pallas_call(
        _tc_in_body,
        grid=(_GRID,),
        in_specs=[_row_spec(x.shape), _full_spec(w1.shape),
                  _row_spec(degp.shape)],
        out_specs=[_row_spec((N, D2)), _row_spec((N, D2))],
        out_shape=[half, half],
    )(x, w1, degp)


def _tc_mid(pp, ga, gb, degp, b1, w2):
    half = jax.ShapeDtypeStruct((N, D2), jnp.float32)
    return pl.pallas_call(
        _tc_mid_body,
        grid=(_GRID,),
        in_specs=[_row_spec(pp.shape), _row_spec(ga.shape),
                  _row_spec(gb.shape), _row_spec(degp.shape),
                  _full_spec(b1.shape), _full_spec(w2.shape)],
        out_specs=[_row_spec((N, D2)), _row_spec((N, D2))],
        out_shape=[half, half],
    )(pp, ga, gb, degp, b1, w2)


def _tc_out(pp, ga, gb, degp, b2, wm1):
    full = jax.ShapeDtypeStruct((N, D), jnp.float32)
    return pl.pallas_call(
        _tc_out_body,
        grid=(_GRID,),
        in_specs=[_row_spec(pp.shape), _row_spec(ga.shape),
                  _row_spec(gb.shape), _row_spec(degp.shape),
                  _full_spec(b2.shape), _full_spec(wm1.shape)],
        out_specs=[_row_spec((N, D)), _row_spec((N, D))],
        out_shape=[full, full],
    )(pp, ga, gb, degp, b2, wm1)


def _tc_head(ape, bm1, wm2, bm2):
    return pl.pallas_call(
        _tc_head_body,
        grid=(NPAIR // _PB,),
        in_specs=[pl.BlockSpec((_PB, D), lambda i: (i, 0)),
                  _full_spec(bm1.shape), _full_spec(wm2.shape),
                  _full_spec(bm2.shape)],
        out_specs=pl.BlockSpec((_PB, 1), lambda i: (i, 0)),
        out_shape=jax.ShapeDtypeStruct((NPAIR, 1), jnp.float32),
    )(ape, bm1, wm2, bm2)


def kernel(x, edge_index, pair, W1, b1, W2, b2, Wm1, bm1, Wm2, bm2):
    src32 = edge_index[0].astype(jnp.int32)
    dst32 = edge_index[1].astype(jnp.int32)
    dst_deg = dst32.reshape(NW, NCE, KE)
    src_prop = src32.reshape(NS, NCP2, KE)
    dst_prop = dst32.reshape(NS, NCP2, KE)
    p0 = pair[:, 0].astype(jnp.int32).reshape(NW, NCP, KP)
    p1 = pair[:, 1].astype(jnp.int32).reshape(NW, NCP, KP)

    degp = _sc_degree(dst_deg)
    g1a, g1b = _tc_in(x, W1, degp)
    pp1 = _sc_prop(g1a, g1b, src_prop, dst_prop)
    g2a, g2b = _tc_mid(pp1, g1a, g1b, degp, b1.reshape(1, D), W2)
    pp2 = _sc_prop(g2a, g2b, src_prop, dst_prop)
    P, Q = _tc_out(pp2, g2a, g2b, degp, b2.reshape(1, D), Wm1)
    ape = _sc_pair(P, Q, p0, p1)
    return _tc_head(ape, bm1.reshape(1, D), Wm2, bm2.reshape(1, 1))


# speedup vs baseline: 18.6069x; 1.0930x over previous
import functools

import jax
import jax.numpy as jnp
from jax import lax
from jax.experimental import pallas as pl
from jax.experimental.pallas import tpu as pltpu
from jax.experimental.pallas import tpu_sc as plsc

N = 10000
D = 128
E = 320000
NPAIR = 65536
NC, NS, NW = 2, 16, 32
D2 = D // 2
EPT = E // NW
EPS = E // NS
KE = 80
NCE = EPT // KE
NCP2 = EPS // KE
CR = 400
CH = N // CR
CK = (CH + NS - 1) // NS + 1
KP = 128
PPT = NPAIR // NW
NCP = PPT // KP


def _mesh():
    return plsc.VectorSubcoreMesh(
        core_axis_name="c", subcore_axis_name="s",
        num_cores=NC, num_subcores=NS)


@functools.partial(
    pl.kernel,
    out_type=jax.ShapeDtypeStruct((NC, N, 16), jnp.float32),
    mesh=_mesh(),
    scratch_types=[
        pltpu.VMEM((NCE, KE), jnp.int32),
        pltpu.VMEM((KE,), jnp.int32),
        pltpu.VMEM((KE, 16), jnp.float32),
        pltpu.VMEM((CR, 16), jnp.float32),
        pltpu.VMEM_SHARED((N, 16), jnp.float32),
    ],
    compiler_params=pltpu.CompilerParams(use_tc_tiling_on_sc=False),
)
def _sc_degree(dst_hbm, degp_hbm, idxbuf, idxchunk, ones, stage, shacc):
    c = lax.axis_index("c")
    s = lax.axis_index("s")
    wid = c * NS + s
    pltpu.sync_copy(dst_hbm.at[wid], idxbuf)

    one16 = jnp.ones((16,), jnp.float32)
    zero16 = jnp.zeros((16,), jnp.float32)

    def fill_ones(i, carry):
        ones[i, :] = one16
        return carry
    lax.fori_loop(0, KE, fill_ones, 0)

    def fill_zero(i, carry):
        stage[i, :] = zero16
        return carry
    lax.fori_loop(0, CR, fill_zero, 0)
    for k in range(CK):
        m = s + k * NS

        @pl.when(m < CH)
        def _():
            pltpu.sync_copy(stage, shacc.at[pl.ds(m * CR, CR), :])
    plsc.subcore_barrier()

    def edge_body(j, carry):
        for u in range(KE // 16):
            idxchunk[pl.ds(u * 16, 16)] = idxbuf[j, pl.ds(u * 16, 16)]
        pltpu.sync_copy(ones, shacc.at[idxchunk], add=True)
        return carry
    lax.fori_loop(0, NCE, edge_body, 0)
    plsc.subcore_barrier()

    for k in range(CK):
        m = s + k * NS

        @pl.when(m < CH)
        def _():
            pltpu.sync_copy(shacc.at[pl.ds(m * CR, CR), :], stage)
            pltpu.sync_copy(stage, degp_hbm.at[c, pl.ds(m * CR, CR), :])


@functools.partial(
    pl.kernel,
    out_type=jax.ShapeDtypeStruct((NC, N, D2), jnp.float32),
    mesh=_mesh(),
    scratch_types=[
        pltpu.VMEM((NCP2, KE), jnp.int32),
        pltpu.VMEM((NCP2, KE), jnp.int32),
        pltpu.VMEM((KE,), jnp.int32),
        pltpu.VMEM((KE,), jnp.int32),
        pltpu.VMEM((KE, D2), jnp.float32),
        pltpu.VMEM((KE, D2), jnp.float32),
        pltpu.VMEM((CR, D2), jnp.float32),
        pltpu.SemaphoreType.DMA,
        pltpu.SemaphoreType.DMA,
        pltpu.VMEM_SHARED((N, D2), jnp.float32),
    ],
    compiler_params=pltpu.CompilerParams(use_tc_tiling_on_sc=False),
)
def _sc_prop(ga_hbm, gb_hbm, src_hbm, dst_hbm, pp_hbm,
             srcbuf, dstbuf, dchunka, dchunkb, rowsa, rowsb, zbuf,
             sema, semb, shacc):
    c = lax.axis_index("c")
    s = lax.axis_index("s")
    pltpu.sync_copy(src_hbm.at[s], srcbuf)
    pltpu.sync_copy(dst_hbm.at[s], dstbuf)

    zero16 = jnp.zeros((16,), jnp.float32)

    def zb(i, carry):
        for u in range(D2 // 16):
            zbuf[i, pl.ds(u * 16, 16)] = zero16
        return carry
    lax.fori_loop(0, CR, zb, 0)
    for k in range(CK):
        m = s + k * NS

        @pl.when(m < CH)
        def _():
            pltpu.sync_copy(zbuf, shacc.at[pl.ds(m * CR, CR), :])
    plsc.subcore_barrier()

    def copy_dst_idx(j, chunk):
        for u in range(KE // 16):
            chunk[pl.ds(u * 16, 16)] = dstbuf[j, pl.ds(u * 16, 16)]

    def run_edges(gsrc):
        copy_dst_idx(0, dchunka)
        pltpu.async_copy(gsrc.at[srcbuf.at[0]], rowsa, sema)

        def body(t, carry):
            j0 = 2 * t
            j1 = j0 + 1
            copy_dst_idx(j1, dchunkb)
            pltpu.async_copy(gsrc.at[srcbuf.at[j1]], rowsb, semb)
            pltpu.make_async_copy(gsrc.at[srcbuf.at[j0]], rowsa, sema).wait()
            pltpu.sync_copy(rowsa, shacc.at[dchunka], add=True)

            @pl.when(t + 1 < NCP2 // 2)
            def _():
                copy_dst_idx(j0 + 2, dchunka)
                pltpu.async_copy(gsrc.at[srcbuf.at[j0 + 2]], rowsa, sema)
            pltpu.make_async_copy(gsrc.at[srcbuf.at[j1]], rowsb, semb).wait()
            pltpu.sync_copy(rowsb, shacc.at[dchunkb], add=True)
            return carry
        lax.fori_loop(0, NCP2 // 2, body, 0)

    @pl.when(c == 0)
    def _():
        run_edges(ga_hbm)

    @pl.when(c == 1)
    def _():
        run_edges(gb_hbm)
    plsc.subcore_barrier()

    for k in range(CK):
        m = s + k * NS

        @pl.when(m < CH)
        def _():
            pltpu.sync_copy(shacc.at[pl.ds(m * CR, CR), :], zbuf)
            pltpu.sync_copy(zbuf, pp_hbm.at[c, pl.ds(m * CR, CR), :])


@functools.partial(
    pl.kernel,
    out_type=[jax.ShapeDtypeStruct((NPAIR, D), jnp.float32),
              jax.ShapeDtypeStruct((NPAIR, D), jnp.float32)],
    mesh=_mesh(),
    scratch_types=[
        pltpu.VMEM((NCP, KP), jnp.int32),
        pltpu.VMEM((NCP, KP), jnp.int32),
        pltpu.VMEM((KP, D), jnp.float32),
        pltpu.VMEM((KP, D), jnp.float32),
        pltpu.VMEM((KP, D), jnp.float32),
        pltpu.VMEM((KP, D), jnp.float32),
        pltpu.SemaphoreType.DMA,
        pltpu.SemaphoreType.DMA,
    ],
    compiler_params=pltpu.CompilerParams(use_tc_tiling_on_sc=False),
)
def _sc_pair(p_hbm, q_hbm, i0_hbm, i1_hbm, ape0_hbm, ape1_hbm,
             i0buf, i1buf, pa, qa, pb, qb, sema, semb):
    c = lax.axis_index("c")
    s = lax.axis_index("s")
    wid = c * NS + s
    pltpu.sync_copy(i0_hbm.at[wid], i0buf)
    pltpu.sync_copy(i1_hbm.at[wid], i1buf)

    base = wid * PPT

    def issue(j, p, q, sem):
        pltpu.async_copy(p_hbm.at[i0buf.at[j]], p, sem)
        pltpu.async_copy(q_hbm.at[i1buf.at[j]], q, sem)

    def waitg(j, p, q, sem):
        pltpu.make_async_copy(p_hbm.at[i0buf.at[j]], p, sem).wait()
        pltpu.make_async_copy(q_hbm.at[i1buf.at[j]], q, sem).wait()

    def outs(j, p, q):
        pltpu.sync_copy(p, ape0_hbm.at[pl.ds(base + j * KP, KP), :])
        pltpu.sync_copy(q, ape1_hbm.at[pl.ds(base + j * KP, KP), :])

    issue(0, pa, qa, sema)
    issue(1, pb, qb, semb)

    def body(t, carry):
        j0 = 2 * t
        j1 = j0 + 1
        waitg(j0, pa, qa, sema)
        outs(j0, pa, qa)

        @pl.when(j0 + 2 < NCP)
        def _():
            issue(j0 + 2, pa, qa, sema)
        waitg(j1, pb, qb, semb)
        outs(j1, pb, qb)

        @pl.when(j1 + 2 < NCP)
        def _():
            issue(j1 + 2, pb, qb, semb)
        return carry
    lax.fori_loop(0, NCP // 2, body, 0)


_RB = 1000
_GRID = N // _RB


def _dinv(degp_ref):
    deg = 1.0 + degp_ref[0, :, 0] + degp_ref[1, :, 0]
    return lax.rsqrt(deg)[:, None]


def _tc_in_body(x_ref, w1_ref, degp_ref, oa_ref, ob_ref):
    g = jnp.dot(x_ref[...], w1_ref[...],
                preferred_element_type=jnp.float32) * _dinv(degp_ref)
    oa_ref[...] = g[:, :D2]
    ob_ref[...] = g[:, D2:]


def _tc_mid_body(pp_ref, ga_ref, gb_ref, degp_ref, b1_ref, w2_ref,
                 oa_ref, ob_ref):
    dinv = _dinv(degp_ref)
    h = jnp.concatenate(
        [pp_ref[0] + ga_ref[...], pp_ref[1] + gb_ref[...]], axis=-1)
    h = jnp.maximum(h * dinv + b1_ref[...], 0.0)
    g = jnp.dot(h, w2_ref[...], preferred_element_type=jnp.float32) * dinv
    oa_ref[...] = g[:, :D2]
    ob_ref[...] = g[:, D2:]


def _tc_out_body(pp_ref, ga_ref, gb_ref, degp_ref, b2_ref, wm1_ref,
                 p_ref, q_ref):
    dinv = _dinv(degp_ref)
    h2 = jnp.concatenate(
        [pp_ref[0] + ga_ref[...], pp_ref[1] + gb_ref[...]], axis=-1)
    h2 = h2 * dinv + b2_ref[...]
    p_ref[...] = jnp.dot(h2, wm1_ref[:D, :],
                         preferred_element_type=jnp.float32)
    q_ref[...] = jnp.dot(h2, wm1_ref[D:, :],
                         preferred_element_type=jnp.float32)


_PB = 2048


def _tc_head_body(ape0_ref, ape1_ref, bm1_ref, wm2_ref, bm2_ref, o_ref):
    t = jnp.maximum(ape0_ref[...] + ape1_ref[...] + bm1_ref[...], 0.0)
    z = jnp.dot(t, wm2_ref[...], preferred_element_type=jnp.float32)
    o_ref[...] = jax.nn.sigmoid(z + bm2_ref[0, 0])


def _row_spec(shape):
    nd = len(shape)
    if nd == 2:
        return pl.BlockSpec((_RB, shape[1]), lambda i: (i, 0))
    return pl.BlockSpec((shape[0], _RB, shape[2]), lambda i: (0, i, 0))


def _full_spec(shape):
    nd = len(shape)
    return pl.BlockSpec(shape, lambda i: (0,) * nd)


def _tc_in(x, w1, degp):
    half = jax.ShapeDtypeStruct((N, D2), jnp.float32)
    return pl.pallas_call(
        _tc_in_body,
        grid=(_GRID,),
        in_specs=[_row_spec(x.shape), _full_spec(w1.shape),
                  _row_spec(degp.shape)],
        out_specs=[_row_spec((N, D2)), _row_spec((N, D2))],
        out_shape=[half, half],
    )(x, w1, degp)


def _tc_mid(pp, ga, gb, degp, b1, w2):
    half = jax.ShapeDtypeStruct((N, D2), jnp.float32)
    return pl.pallas_call(
        _tc_mid_body,
        grid=(_GRID,),
        in_specs=[_row_spec(pp.shape), _row_spec(ga.shape),
                  _row_spec(gb.shape), _row_spec(degp.shape),
                  _full_spec(b1.shape), _full_spec(w2.shape)],
        out_specs=[_row_spec((N, D2)), _row_spec((N, D2))],
        out_shape=[half, half],
    )(pp, ga, gb, degp, b1, w2)


def _tc_out(pp, ga, gb, degp, b2, wm1):
    full = jax.ShapeDtypeStruct((N, D), jnp.float32)
    return pl.pallas_call(
        _tc_out_body,
        grid=(_GRID,),
        in_specs=[_row_spec(pp.shape), _row_spec(ga.shape),
                  _row_spec(gb.shape), _row_spec(degp.shape),
                  _full_spec(b2.shape), _full_spec(wm1.shape)],
        out_specs=[_row_spec((N, D)), _row_spec((N, D))],
        out_shape=[full, full],
    )(pp, ga, gb, degp, b2, wm1)


def _tc_head(ape0, ape1, bm1, wm2, bm2):
    return pl.pallas_call(
        _tc_head_body,
        grid=(NPAIR // _PB,),
        in_specs=[pl.BlockSpec((_PB, D), lambda i: (i, 0)),
                  pl.BlockSpec((_PB, D), lambda i: (i, 0)),
                  _full_spec(bm1.shape), _full_spec(wm2.shape),
                  _full_spec(bm2.shape)],
        out_specs=pl.BlockSpec((_PB, 1), lambda i: (i, 0)),
        out_shape=jax.ShapeDtypeStruct((NPAIR, 1), jnp.float32),
    )(ape0, ape1, bm1, wm2, bm2)


def kernel(x, edge_index, pair, W1, b1, W2, b2, Wm1, bm1, Wm2, bm2):
    src32 = edge_index[0].astype(jnp.int32)
    dst32 = edge_index[1].astype(jnp.int32)
    dst_deg = dst32.reshape(NW, NCE, KE)
    src_prop = src32.reshape(NS, NCP2, KE)
    dst_prop = dst32.reshape(NS, NCP2, KE)
    p0 = pair[:, 0].astype(jnp.int32).reshape(NW, NCP, KP)
    p1 = pair[:, 1].astype(jnp.int32).reshape(NW, NCP, KP)

    degp = _sc_degree(dst_deg)
    g1a, g1b = _tc_in(x, W1, degp)
    pp1 = _sc_prop(g1a, g1b, src_prop, dst_prop)
    g2a, g2b = _tc_mid(pp1, g1a, g1b, degp, b1.reshape(1, D), W2)
    pp2 = _sc_prop(g2a, g2b, src_prop, dst_prop)
    P, Q = _tc_out(pp2, g2a, g2b, degp, b2.reshape(1, D), Wm1)
    ape0, ape1 = _sc_pair(P, Q, p0, p1)
    return _tc_head(ape0, ape1, bm1.reshape(1, D), Wm2, bm2.reshape(1, 1))


# speedup vs baseline: 20.7904x; 1.1173x over previous
import functools

import jax
import jax.numpy as jnp
from jax import lax
from jax.experimental import pallas as pl
from jax.experimental.pallas import tpu as pltpu
from jax.experimental.pallas import tpu_sc as plsc

N = 10000
D = 128
E = 320000
NPAIR = 65536
NC, NS, NW = 2, 16, 32
D2 = D // 2
EPT = E // NW
EPS = E // NS
KE = 80
NCE = EPT // KE
NCP2 = EPS // KE
CR = 400
CH = N // CR
CK = (CH + NS - 1) // NS + 1
KP = 128
PPT = NPAIR // NW
NCP = PPT // KP


def _mesh():
    return plsc.VectorSubcoreMesh(
        core_axis_name="c", subcore_axis_name="s",
        num_cores=NC, num_subcores=NS)


@functools.partial(
    pl.kernel,
    out_type=jax.ShapeDtypeStruct((NC, N, 16), jnp.float32),
    mesh=_mesh(),
    scratch_types=[
        pltpu.VMEM((NCE, KE), jnp.int32),
        pltpu.VMEM((KE,), jnp.int32),
        pltpu.VMEM((KE, 16), jnp.float32),
        pltpu.VMEM((CR, 16), jnp.float32),
        pltpu.VMEM_SHARED((N, 16), jnp.float32),
    ],
    compiler_params=pltpu.CompilerParams(use_tc_tiling_on_sc=False),
)
def _sc_degree(dst_hbm, degp_hbm, idxbuf, idxchunk, ones, stage, shacc):
    c = lax.axis_index("c")
    s = lax.axis_index("s")
    wid = c * NS + s
    pltpu.sync_copy(dst_hbm.at[wid], idxbuf)

    one16 = jnp.ones((16,), jnp.float32)
    zero16 = jnp.zeros((16,), jnp.float32)

    def fill_ones(i, carry):
        ones[i, :] = one16
        return carry
    lax.fori_loop(0, KE, fill_ones, 0)

    def fill_zero(i, carry):
        stage[i, :] = zero16
        return carry
    lax.fori_loop(0, CR, fill_zero, 0)
    for k in range(CK):
        m = s + k * NS

        @pl.when(m < CH)
        def _():
            pltpu.sync_copy(stage, shacc.at[pl.ds(m * CR, CR), :])
    plsc.subcore_barrier()

    def edge_body(j, carry):
        for u in range(KE // 16):
            idxchunk[pl.ds(u * 16, 16)] = idxbuf[j, pl.ds(u * 16, 16)]
        pltpu.sync_copy(ones, shacc.at[idxchunk], add=True)
        return carry
    lax.fori_loop(0, NCE, edge_body, 0)
    plsc.subcore_barrier()

    for k in range(CK):
        m = s + k * NS

        @pl.when(m < CH)
        def _():
            pltpu.sync_copy(shacc.at[pl.ds(m * CR, CR), :], stage)
            pltpu.sync_copy(stage, degp_hbm.at[c, pl.ds(m * CR, CR), :])


KF = 128
NKF = EPS // KF
KT = EPS - NKF * KF


@functools.partial(
    pl.kernel,
    out_type=jax.ShapeDtypeStruct((NC, N, D2), jnp.float32),
    mesh=_mesh(),
    scratch_types=[
        pltpu.VMEM((EPS,), jnp.int32),
        pltpu.VMEM((EPS,), jnp.int32),
        pltpu.VMEM((KF,), jnp.int32),
        pltpu.VMEM((KF,), jnp.int32),
        pltpu.VMEM((KF, D2), jnp.float32),
        pltpu.VMEM((KF, D2), jnp.float32),
        pltpu.VMEM((KT,), jnp.int32),
        pltpu.VMEM((KT, D2), jnp.float32),
        pltpu.VMEM((CR, D2), jnp.float32),
        pltpu.SemaphoreType.DMA,
        pltpu.SemaphoreType.DMA,
        pltpu.VMEM_SHARED((N, D2), jnp.float32),
    ],
    compiler_params=pltpu.CompilerParams(use_tc_tiling_on_sc=False),
)
def _sc_prop(ga_hbm, gb_hbm, src_hbm, dst_hbm, pp_hbm,
             srcbuf, dstbuf, dchunka, dchunkb, rowsa, rowsb, dchunkt, rowst,
             zbuf, sema, semb, shacc):
    c = lax.axis_index("c")
    s = lax.axis_index("s")
    pltpu.sync_copy(src_hbm.at[s], srcbuf)
    pltpu.sync_copy(dst_hbm.at[s], dstbuf)

    zero16 = jnp.zeros((16,), jnp.float32)

    def zb(i, carry):
        for u in range(D2 // 16):
            zbuf[i, pl.ds(u * 16, 16)] = zero16
        return carry
    lax.fori_loop(0, CR, zb, 0)
    for k in range(CK):
        m = s + k * NS

        @pl.when(m < CH)
        def _():
            pltpu.sync_copy(zbuf, shacc.at[pl.ds(m * CR, CR), :])
    plsc.subcore_barrier()

    def copy_dst_idx(j, chunk, width):
        for u in range(width // 16):
            chunk[pl.ds(u * 16, 16)] = dstbuf[pl.ds(j * KF + u * 16, 16)]

    def run_edges(gsrc):
        def issue(j, rows, sem):
            pltpu.async_copy(gsrc.at[srcbuf.at[pl.ds(j * KF, KF)]], rows, sem)

        def waitg(j, rows, sem):
            pltpu.make_async_copy(
                gsrc.at[srcbuf.at[pl.ds(j * KF, KF)]], rows, sem).wait()

        copy_dst_idx(0, dchunka, KF)
        issue(0, rowsa, sema)

        def body(t, carry):
            j0 = 2 * t
            j1 = j0 + 1
            copy_dst_idx(j1, dchunkb, KF)
            issue(j1, rowsb, semb)
            waitg(j0, rowsa, sema)
            pltpu.sync_copy(rowsa, shacc.at[dchunka], add=True)

            @pl.when(t + 1 < NKF // 2)
            def _():
                copy_dst_idx(j0 + 2, dchunka, KF)
                issue(j0 + 2, rowsa, sema)
            waitg(j1, rowsb, semb)
            pltpu.sync_copy(rowsb, shacc.at[dchunkb], add=True)
            return carry
        lax.fori_loop(0, NKF // 2, body, 0)
        for u in range(KT // 16):
            dchunkt[pl.ds(u * 16, 16)] = dstbuf[pl.ds(NKF * KF + u * 16, 16)]
        pltpu.async_copy(
            gsrc.at[srcbuf.at[pl.ds(NKF * KF, KT)]], rowst, sema).wait()
        pltpu.sync_copy(rowst, shacc.at[dchunkt], add=True)

    @pl.when(c == 0)
    def _():
        run_edges(ga_hbm)

    @pl.when(c == 1)
    def _():
        run_edges(gb_hbm)
    plsc.subcore_barrier()

    for k in range(CK):
        m = s + k * NS

        @pl.when(m < CH)
        def _():
            pltpu.sync_copy(shacc.at[pl.ds(m * CR, CR), :], zbuf)
            pltpu.sync_copy(zbuf, pp_hbm.at[c, pl.ds(m * CR, CR), :])


@functools.partial(
    pl.kernel,
    out_type=[jax.ShapeDtypeStruct((NPAIR, D), jnp.float32),
              jax.ShapeDtypeStruct((NPAIR, D), jnp.float32)],
    mesh=_mesh(),
    scratch_types=[
        pltpu.VMEM((NCP, KP), jnp.int32),
        pltpu.VMEM((NCP, KP), jnp.int32),
        pltpu.VMEM((KP, D), jnp.float32),
        pltpu.VMEM((KP, D), jnp.float32),
        pltpu.VMEM((KP, D), jnp.float32),
        pltpu.VMEM((KP, D), jnp.float32),
        pltpu.SemaphoreType.DMA,
        pltpu.SemaphoreType.DMA,
    ],
    compiler_params=pltpu.CompilerParams(use_tc_tiling_on_sc=False),
)
def _sc_pair(p_hbm, q_hbm, i0_hbm, i1_hbm, ape0_hbm, ape1_hbm,
             i0buf, i1buf, pa, qa, pb, qb, sema, semb):
    c = lax.axis_index("c")
    s = lax.axis_index("s")
    wid = c * NS + s
    pltpu.sync_copy(i0_hbm.at[wid], i0buf)
    pltpu.sync_copy(i1_hbm.at[wid], i1buf)

    base = wid * PPT

    def issue(j, p, q, sem):
        pltpu.async_copy(p_hbm.at[i0buf.at[j]], p, sem)
        pltpu.async_copy(q_hbm.at[i1buf.at[j]], q, sem)

    def waitg(j, p, q, sem):
        pltpu.make_async_copy(p_hbm.at[i0buf.at[j]], p, sem).wait()
        pltpu.make_async_copy(q_hbm.at[i1buf.at[j]], q, sem).wait()

    def outs(j, p, q):
        pltpu.sync_copy(p, ape0_hbm.at[pl.ds(base + j * KP, KP), :])
        pltpu.sync_copy(q, ape1_hbm.at[pl.ds(base + j * KP, KP), :])

    issue(0, pa, qa, sema)
    issue(1, pb, qb, semb)

    def body(t, carry):
        j0 = 2 * t
        j1 = j0 + 1
        waitg(j0, pa, qa, sema)
        outs(j0, pa, qa)

        @pl.when(j0 + 2 < NCP)
        def _():
            issue(j0 + 2, pa, qa, sema)
        waitg(j1, pb, qb, semb)
        outs(j1, pb, qb)

        @pl.when(j1 + 2 < NCP)
        def _():
            issue(j1 + 2, pb, qb, semb)
        return carry
    lax.fori_loop(0, NCP // 2, body, 0)


_RB = 1000
_GRID = N // _RB


def _dinv(degp_ref):
    deg = 1.0 + degp_ref[0, :, 0] + degp_ref[1, :, 0]
    return lax.rsqrt(deg)[:, None]


def _tc_in_body(x_ref, w1_ref, degp_ref, oa_ref, ob_ref):
    g = jnp.dot(x_ref[...], w1_ref[...],
                preferred_element_type=jnp.float32) * _dinv(degp_ref)
    oa_ref[...] = g[:, :D2]
    ob_ref[...] = g[:, D2:]


def _tc_mid_body(pp_ref, ga_ref, gb_ref, degp_ref, b1_ref, w2_ref,
                 oa_ref, ob_ref):
    dinv = _dinv(degp_ref)
    h = jnp.concatenate(
        [pp_ref[0] + ga_ref[...], pp_ref[1] + gb_ref[...]], axis=-1)
    h = jnp.maximum(h * dinv + b1_ref[...], 0.0)
    g = jnp.dot(h, w2_ref[...], preferred_element_type=jnp.float32) * dinv
    oa_ref[...] = g[:, :D2]
    ob_ref[...] = g[:, D2:]


def _tc_out_body(pp_ref, ga_ref, gb_ref, degp_ref, b2_ref, wm1_ref,
                 p_ref, q_ref):
    dinv = _dinv(degp_ref)
    h2 = jnp.concatenate(
        [pp_ref[0] + ga_ref[...], pp_ref[1] + gb_ref[...]], axis=-1)
    h2 = h2 * dinv + b2_ref[...]
    p_ref[...] = jnp.dot(h2, wm1_ref[:D, :],
                         preferred_element_type=jnp.float32)
    q_ref[...] = jnp.dot(h2, wm1_ref[D:, :],
                         preferred_element_type=jnp.float32)


_PB = 2048


def _tc_head_body(ape0_ref, ape1_ref, bm1_ref, wm2_ref, bm2_ref, o_ref):
    t = jnp.maximum(ape0_ref[...] + ape1_ref[...] + bm1_ref[...], 0.0)
    z = jnp.dot(t, wm2_ref[...], preferred_element_type=jnp.float32)
    o_ref[...] = jax.nn.sigmoid(z + bm2_ref[0, 0])


def _row_spec(shape):
    nd = len(shape)
    if nd == 2:
        return pl.BlockSpec((_RB, shape[1]), lambda i: (i, 0))
    return pl.BlockSpec((shape[0], _RB, shape[2]), lambda i: (0, i, 0))


def _full_spec(shape):
    nd = len(shape)
    return pl.BlockSpec(shape, lambda i: (0,) * nd)


def _tc_in(x, w1, degp):
    half = jax.ShapeDtypeStruct((N, D2), jnp.float32)
    return pl.pallas_call(
        _tc_in_body,
        grid=(_GRID,),
        in_specs=[_row_spec(x.shape), _full_spec(w1.shape),
                  _row_spec(degp.shape)],
        out_specs=[_row_spec((N, D2)), _row_spec((N, D2))],
        out_shape=[half, half],
    )(x, w1, degp)


def _tc_mid(pp, ga, gb, degp, b1, w2):
    half = jax.ShapeDtypeStruct((N, D2), jnp.float32)
    return pl.pallas_call(
        _tc_mid_body,
        grid=(_GRID,),
        in_specs=[_row_spec(pp.shape), _row_spec(ga.shape),
                  _row_spec(gb.shape), _row_spec(degp.shape),
                  _full_spec(b1.shape), _full_spec(w2.shape)],
        out_specs=[_row_spec((N, D2)), _row_spec((N, D2))],
        out_shape=[half, half],
    )(pp, ga, gb, degp, b1, w2)


def _tc_out(pp, ga, gb, degp, b2, wm1):
    full = jax.ShapeDtypeStruct((N, D), jnp.float32)
    return pl.pallas_call(
        _tc_out_body,
        grid=(_GRID,),
        in_specs=[_row_spec(pp.shape), _row_spec(ga.shape),
                  _row_spec(gb.shape), _row_spec(degp.shape),
                  _full_spec(b2.shape), _full_spec(wm1.shape)],
        out_specs=[_row_spec((N, D)), _row_spec((N, D))],
        out_shape=[full, full],
    )(pp, ga, gb, degp, b2, wm1)


def _tc_head(ape0, ape1, bm1, wm2, bm2):
    return pl.pallas_call(
        _tc_head_body,
        grid=(NPAIR // _PB,),
        in_specs=[pl.BlockSpec((_PB, D), lambda i: (i, 0)),
                  pl.BlockSpec((_PB, D), lambda i: (i, 0)),
                  _full_spec(bm1.shape), _full_spec(wm2.shape),
                  _full_spec(bm2.shape)],
        out_specs=pl.BlockSpec((_PB, 1), lambda i: (i, 0)),
        out_shape=jax.ShapeDtypeStruct((NPAIR, 1), jnp.float32),
    )(ape0, ape1, bm1, wm2, bm2)


def kernel(x, edge_index, pair, W1, b1, W2, b2, Wm1, bm1, Wm2, bm2):
    src32 = edge_index[0].astype(jnp.int32)
    dst32 = edge_index[1].astype(jnp.int32)
    dst_deg = dst32.reshape(NW, NCE, KE)
    src_prop = src32.reshape(NS, EPS)
    dst_prop = dst32.reshape(NS, EPS)
    p0 = pair[:, 0].astype(jnp.int32).reshape(NW, NCP, KP)
    p1 = pair[:, 1].astype(jnp.int32).reshape(NW, NCP, KP)

    degp = _sc_degree(dst_deg)
    g1a, g1b = _tc_in(x, W1, degp)
    pp1 = _sc_prop(g1a, g1b, src_prop, dst_prop)
    g2a, g2b = _tc_mid(pp1, g1a, g1b, degp, b1.reshape(1, D), W2)
    pp2 = _sc_prop(g2a, g2b, src_prop, dst_prop)
    P, Q = _tc_out(pp2, g2a, g2b, degp, b2.reshape(1, D), Wm1)
    ape0, ape1 = _sc_pair(P, Q, p0, p1)
    return _tc_head(ape0, ape1, bm1.reshape(1, D), Wm2, bm2.reshape(1, 1))


# speedup vs baseline: 20.9968x; 1.0099x over previous
import functools

import jax
import jax.numpy as jnp
from jax import lax
from jax.experimental import pallas as pl
from jax.experimental.pallas import tpu as pltpu
from jax.experimental.pallas import tpu_sc as plsc

N = 10000
D = 128
E = 320000
NPAIR = 65536
NC, NS, NW = 2, 16, 32
D2 = D // 2
EPT = E // NW
EPS = E // NS
KE = 80
NCE = EPT // KE
NCP2 = EPS // KE
CR = 400
CH = N // CR
CK = (CH + NS - 1) // NS + 1
KP = 128
PPT = NPAIR // NW
NCP = PPT // KP


def _mesh():
    return plsc.VectorSubcoreMesh(
        core_axis_name="c", subcore_axis_name="s",
        num_cores=NC, num_subcores=NS)


@functools.partial(
    pl.kernel,
    out_type=jax.ShapeDtypeStruct((NC, N, 16), jnp.float32),
    mesh=_mesh(),
    scratch_types=[
        pltpu.VMEM((NCE, KE), jnp.int32),
        pltpu.VMEM((KE,), jnp.int32),
        pltpu.VMEM((KE, 16), jnp.float32),
        pltpu.VMEM((CR, 16), jnp.float32),
        pltpu.VMEM_SHARED((N, 16), jnp.float32),
    ],
    compiler_params=pltpu.CompilerParams(use_tc_tiling_on_sc=False),
)
def _sc_degree(dst_hbm, degp_hbm, idxbuf, idxchunk, ones, stage, shacc):
    c = lax.axis_index("c")
    s = lax.axis_index("s")
    wid = c * NS + s
    pltpu.sync_copy(dst_hbm.at[wid], idxbuf)

    one16 = jnp.ones((16,), jnp.float32)
    zero16 = jnp.zeros((16,), jnp.float32)

    def fill_ones(i, carry):
        ones[i, :] = one16
        return carry
    lax.fori_loop(0, KE, fill_ones, 0)

    def fill_zero(i, carry):
        stage[i, :] = zero16
        return carry
    lax.fori_loop(0, CR, fill_zero, 0)
    for k in range(CK):
        m = s + k * NS

        @pl.when(m < CH)
        def _():
            pltpu.sync_copy(stage, shacc.at[pl.ds(m * CR, CR), :])
    plsc.subcore_barrier()

    def edge_body(j, carry):
        for u in range(KE // 16):
            idxchunk[pl.ds(u * 16, 16)] = idxbuf[j, pl.ds(u * 16, 16)]
        pltpu.sync_copy(ones, shacc.at[idxchunk], add=True)
        return carry
    lax.fori_loop(0, NCE, edge_body, 0)
    plsc.subcore_barrier()

    for k in range(CK):
        m = s + k * NS

        @pl.when(m < CH)
        def _():
            pltpu.sync_copy(shacc.at[pl.ds(m * CR, CR), :], stage)
            pltpu.sync_copy(stage, degp_hbm.at[c, pl.ds(m * CR, CR), :])


KF = 128
NKF = EPS // KF
KT = EPS - NKF * KF


@functools.partial(
    pl.kernel,
    out_type=jax.ShapeDtypeStruct((NC, N, D2), jnp.float32),
    mesh=_mesh(),
    scratch_types=[
        pltpu.VMEM((EPS,), jnp.int32),
        pltpu.VMEM((EPS,), jnp.int32),
        pltpu.VMEM((2, KF), jnp.int32),
        pltpu.VMEM((2, KF, D2), jnp.float32),
        pltpu.VMEM((KT,), jnp.int32),
        pltpu.VMEM((KT, D2), jnp.float32),
        pltpu.VMEM((CR, D2), jnp.float32),
        pltpu.SemaphoreType.DMA((2,)),
        pltpu.VMEM_SHARED((N, D2), jnp.float32),
    ],
    compiler_params=pltpu.CompilerParams(use_tc_tiling_on_sc=False),
)
def _sc_prop(ga_hbm, gb_hbm, src_hbm, dst_hbm, pp_hbm,
             srcbuf, dstbuf, dchbuf, rowsbuf, dchunkt, rowst, zbuf,
             sems, shacc):
    c = lax.axis_index("c")
    s = lax.axis_index("s")
    pltpu.sync_copy(src_hbm.at[s], srcbuf)
    pltpu.sync_copy(dst_hbm.at[s], dstbuf)

    zero16 = jnp.zeros((16,), jnp.float32)

    def zb(i, carry):
        for u in range(D2 // 16):
            zbuf[i, pl.ds(u * 16, 16)] = zero16
        return carry
    lax.fori_loop(0, CR, zb, 0)
    for k in range(CK):
        m = s + k * NS

        @pl.when(m < CH)
        def _():
            pltpu.sync_copy(zbuf, shacc.at[pl.ds(m * CR, CR), :])
    plsc.subcore_barrier()

    def run_edges(gsrc):
        def copyidx(j, slv):
            for u in range(KF // 16):
                dchbuf[slv, pl.ds(u * 16, 16)] = (
                    dstbuf[pl.ds(j * KF + u * 16, 16)])

        def gissue(j, slv):
            pltpu.async_copy(gsrc.at[srcbuf.at[pl.ds(j * KF, KF)]],
                             rowsbuf.at[slv], sems.at[slv])

        def gwait(j, slv):
            pltpu.make_async_copy(gsrc.at[srcbuf.at[pl.ds(j * KF, KF)]],
                                  rowsbuf.at[slv], sems.at[slv]).wait()

        def scat(slv):
            pltpu.sync_copy(rowsbuf.at[slv], shacc.at[dchbuf.at[slv]],
                            add=True)

        copyidx(0, 0)
        gissue(0, 0)

        def body(j, carry):
            slv = lax.rem(j, 2)
            copyidx(j, slv)
            gissue(j, slv)
            slv2 = lax.rem(j + 1, 2)
            gwait(j - 1, slv2)
            scat(slv2)
            return carry
        lax.fori_loop(1, NKF, body, 0)
        gwait(NKF - 1, lax.rem(NKF - 1, 2))
        scat(lax.rem(NKF - 1, 2))

        for u in range(KT // 16):
            dchunkt[pl.ds(u * 16, 16)] = dstbuf[pl.ds(NKF * KF + u * 16, 16)]
        pltpu.async_copy(
            gsrc.at[srcbuf.at[pl.ds(NKF * KF, KT)]], rowst, sems.at[0]).wait()
        pltpu.sync_copy(rowst, shacc.at[dchunkt], add=True)

    @pl.when(c == 0)
    def _():
        run_edges(ga_hbm)

    @pl.when(c == 1)
    def _():
        run_edges(gb_hbm)
    plsc.subcore_barrier()

    for k in range(CK):
        m = s + k * NS

        @pl.when(m < CH)
        def _():
            pltpu.sync_copy(shacc.at[pl.ds(m * CR, CR), :], zbuf)
            pltpu.sync_copy(zbuf, pp_hbm.at[c, pl.ds(m * CR, CR), :])


@functools.partial(
    pl.kernel,
    out_type=[jax.ShapeDtypeStruct((NPAIR, D), jnp.float32),
              jax.ShapeDtypeStruct((NPAIR, D), jnp.float32)],
    mesh=_mesh(),
    scratch_types=[
        pltpu.VMEM((NCP, KP), jnp.int32),
        pltpu.VMEM((NCP, KP), jnp.int32),
        pltpu.VMEM((KP, D), jnp.float32),
        pltpu.VMEM((KP, D), jnp.float32),
        pltpu.VMEM((KP, D), jnp.float32),
        pltpu.VMEM((KP, D), jnp.float32),
        pltpu.SemaphoreType.DMA,
        pltpu.SemaphoreType.DMA,
    ],
)
def _sc_pair(p_hbm, q_hbm, i0_hbm, i1_hbm, ape0_hbm, ape1_hbm,
             i0buf, i1buf, pa, qa, pb, qb, sema, semb):
    c = lax.axis_index("c")
    s = lax.axis_index("s")
    wid = c * NS + s
    pltpu.sync_copy(i0_hbm.at[wid], i0buf)
    pltpu.sync_copy(i1_hbm.at[wid], i1buf)

    base = wid * PPT

    def issue(j, p, q, sem):
        pltpu.async_copy(p_hbm.at[i0buf.at[j]], p, sem)
        pltpu.async_copy(q_hbm.at[i1buf.at[j]], q, sem)

    def waitg(j, p, q, sem):
        pltpu.make_async_copy(p_hbm.at[i0buf.at[j]], p, sem).wait()
        pltpu.make_async_copy(q_hbm.at[i1buf.at[j]], q, sem).wait()

    def outs(j, p, q):
        pltpu.sync_copy(p, ape0_hbm.at[pl.ds(base + j * KP, KP), :])
        pltpu.sync_copy(q, ape1_hbm.at[pl.ds(base + j * KP, KP), :])

    issue(0, pa, qa, sema)
    issue(1, pb, qb, semb)

    def body(t, carry):
        j0 = 2 * t
        j1 = j0 + 1
        waitg(j0, pa, qa, sema)
        outs(j0, pa, qa)

        @pl.when(j0 + 2 < NCP)
        def _():
            issue(j0 + 2, pa, qa, sema)
        waitg(j1, pb, qb, semb)
        outs(j1, pb, qb)

        @pl.when(j1 + 2 < NCP)
        def _():
            issue(j1 + 2, pb, qb, semb)
        return carry
    lax.fori_loop(0, NCP // 2, body, 0)


_RB = 2000
_GRID = N // _RB


def _dinv(degp_ref):
    deg = 1.0 + degp_ref[0, :, 0] + degp_ref[1, :, 0]
    return lax.rsqrt(deg)[:, None]


def _tc_in_body(x_ref, w1_ref, degp_ref, oa_ref, ob_ref):
    g = jnp.dot(x_ref[...], w1_ref[...],
                preferred_element_type=jnp.float32) * _dinv(degp_ref)
    oa_ref[...] = g[:, :D2]
    ob_ref[...] = g[:, D2:]


def _tc_mid_body(pp_ref, ga_ref, gb_ref, degp_ref, b1_ref, w2_ref,
                 oa_ref, ob_ref):
    dinv = _dinv(degp_ref)
    h = jnp.concatenate(
        [pp_ref[0] + ga_ref[...], pp_ref[1] + gb_ref[...]], axis=-1)
    h = jnp.maximum(h * dinv + b1_ref[...], 0.0)
    g = jnp.dot(h, w2_ref[...], preferred_element_type=jnp.float32) * dinv
    oa_ref[...] = g[:, :D2]
    ob_ref[...] = g[:, D2:]


def _tc_out_body(pp_ref, ga_ref, gb_ref, degp_ref, b2_ref, wm1_ref,
                 p_ref, q_ref):
    dinv = _dinv(degp_ref)
    h2 = jnp.concatenate(
        [pp_ref[0] + ga_ref[...], pp_ref[1] + gb_ref[...]], axis=-1)
    h2 = h2 * dinv + b2_ref[...]
    p_ref[...] = jnp.dot(h2, wm1_ref[:D, :],
                         preferred_element_type=jnp.float32)
    q_ref[...] = jnp.dot(h2, wm1_ref[D:, :],
                         preferred_element_type=jnp.float32)


_PB = 2048


def _tc_head_body(ape0_ref, ape1_ref, bm1_ref, wm2_ref, bm2_ref, o_ref):
    t = jnp.maximum(ape0_ref[...] + ape1_ref[...] + bm1_ref[...], 0.0)
    z = jnp.dot(t, wm2_ref[...], preferred_element_type=jnp.float32)
    o_ref[...] = jax.nn.sigmoid(z + bm2_ref[0, 0])


def _row_spec(shape):
    nd = len(shape)
    if nd == 2:
        return pl.BlockSpec((_RB, shape[1]), lambda i: (i, 0))
    return pl.BlockSpec((shape[0], _RB, shape[2]), lambda i: (0, i, 0))


def _full_spec(shape):
    nd = len(shape)
    return pl.BlockSpec(shape, lambda i: (0,) * nd)


def _tc_in(x, w1, degp):
    half = jax.ShapeDtypeStruct((N, D2), jnp.float32)
    return pl.pallas_call(
        _tc_in_body,
        grid=(_GRID,),
        in_specs=[_row_spec(x.shape), _full_spec(w1.shape),
                  _row_spec(degp.shape)],
        out_specs=[_row_spec((N, D2)), _row_spec((N, D2))],
        out_shape=[half, half],
    )(x, w1, degp)


def _tc_mid(pp, ga, gb, degp, b1, w2):
    half = jax.ShapeDtypeStruct((N, D2), jnp.float32)
    return pl.pallas_call(
        _tc_mid_body,
        grid=(_GRID,),
        in_specs=[_row_spec(pp.shape), _row_spec(ga.shape),
                  _row_spec(gb.shape), _row_spec(degp.shape),
                  _full_spec(b1.shape), _full_spec(w2.shape)],
        out_specs=[_row_spec((N, D2)), _row_spec((N, D2))],
        out_shape=[half, half],
    )(pp, ga, gb, degp, b1, w2)


def _tc_out(pp, ga, gb, degp, b2, wm1):
    full = jax.ShapeDtypeStruct((N, D), jnp.float32)
    return pl.pallas_call(
        _tc_out_body,
        grid=(_GRID,),
        in_specs=[_row_spec(pp.shape), _row_spec(ga.shape),
                  _row_spec(gb.shape), _row_spec(degp.shape),
                  _full_spec(b2.shape), _full_spec(wm1.shape)],
        out_specs=[_row_spec((N, D)), _row_spec((N, D))],
        out_shape=[full, full],
    )(pp, ga, gb, degp, b2, wm1)


def _tc_head(ape0, ape1, bm1, wm2, bm2):
    return pl.pallas_call(
        _tc_head_body,
        grid=(NPAIR // _PB,),
        in_specs=[pl.BlockSpec((_PB, D), lambda i: (i, 0)),
                  pl.BlockSpec((_PB, D), lambda i: (i, 0)),
                  _full_spec(bm1.shape), _full_spec(wm2.shape),
                  _full_spec(bm2.shape)],
        out_specs=pl.BlockSpec((_PB, 1), lambda i: (i, 0)),
        out_shape=jax.ShapeDtypeStruct((NPAIR, 1), jnp.float32),
    )(ape0, ape1, bm1, wm2, bm2)


def kernel(x, edge_index, pair, W1, b1, W2, b2, Wm1, bm1, Wm2, bm2):
    src32 = edge_index[0].astype(jnp.int32)
    dst32 = edge_index[1].astype(jnp.int32)
    dst_deg = dst32.reshape(NW, NCE, KE)
    src_prop = src32.reshape(NS, EPS)
    dst_prop = dst32.reshape(NS, EPS)
    p0 = pair[:, 0].astype(jnp.int32).reshape(NW, NCP, KP)
    p1 = pair[:, 1].astype(jnp.int32).reshape(NW, NCP, KP)

    degp = _sc_degree(dst_deg)
    g1a, g1b = _tc_in(x, W1, degp)
    pp1 = _sc_prop(g1a, g1b, src_prop, dst_prop)
    g2a, g2b = _tc_mid(pp1, g1a, g1b, degp, b1.reshape(1, D), W2)
    pp2 = _sc_prop(g2a, g2b, src_prop, dst_prop)
    P, Q = _tc_out(pp2, g2a, g2b, degp, b2.reshape(1, D), Wm1)
    ape0, ape1 = _sc_pair(P, Q, p0, p1)
    return _tc_head(ape0, ape1, bm1.reshape(1, D), Wm2, bm2.reshape(1, 1))


# speedup vs baseline: 22.4452x; 1.0690x over previous
import functools

import jax
import jax.numpy as jnp
from jax import lax
from jax.experimental import pallas as pl
from jax.experimental.pallas import tpu as pltpu
from jax.experimental.pallas import tpu_sc as plsc

N = 10000
D = 128
E = 320000
NPAIR = 65536
NC, NS, NW = 2, 16, 32
D2 = D // 2
EPT = E // NW
EPS = E // NS
KE = 80
NCE = EPT // KE
NCP2 = EPS // KE
CR = 400
CH = N // CR
CK = (CH + NS - 1) // NS + 1
KP = 128
PPT = NPAIR // NW
NCP = PPT // KP


def _mesh():
    return plsc.VectorSubcoreMesh(
        core_axis_name="c", subcore_axis_name="s",
        num_cores=NC, num_subcores=NS)


@functools.partial(
    pl.kernel,
    out_type=jax.ShapeDtypeStruct((NC, N, 16), jnp.float32),
    mesh=_mesh(),
    scratch_types=[
        pltpu.VMEM((NCE, KE), jnp.int32),
        pltpu.VMEM((KE,), jnp.int32),
        pltpu.VMEM((KE, 16), jnp.float32),
        pltpu.VMEM((CR, 16), jnp.float32),
        pltpu.VMEM_SHARED((N, 16), jnp.float32),
    ],
    compiler_params=pltpu.CompilerParams(use_tc_tiling_on_sc=False),
)
def _sc_degree(dst_hbm, degp_hbm, idxbuf, idxchunk, ones, stage, shacc):
    c = lax.axis_index("c")
    s = lax.axis_index("s")
    wid = c * NS + s
    pltpu.sync_copy(dst_hbm.at[wid], idxbuf)

    one16 = jnp.ones((16,), jnp.float32)
    zero16 = jnp.zeros((16,), jnp.float32)

    def fill_ones(i, carry):
        ones[i, :] = one16
        return carry
    lax.fori_loop(0, KE, fill_ones, 0)

    def fill_zero(i, carry):
        stage[i, :] = zero16
        return carry
    lax.fori_loop(0, CR, fill_zero, 0)
    for k in range(CK):
        m = s + k * NS

        @pl.when(m < CH)
        def _():
            pltpu.sync_copy(stage, shacc.at[pl.ds(m * CR, CR), :])
    plsc.subcore_barrier()

    def edge_body(j, carry):
        for u in range(KE // 16):
            idxchunk[pl.ds(u * 16, 16)] = idxbuf[j, pl.ds(u * 16, 16)]
        pltpu.sync_copy(ones, shacc.at[idxchunk], add=True)
        return carry
    lax.fori_loop(0, NCE, edge_body, 0)
    plsc.subcore_barrier()

    for k in range(CK):
        m = s + k * NS

        @pl.when(m < CH)
        def _():
            pltpu.sync_copy(shacc.at[pl.ds(m * CR, CR), :], stage)
            pltpu.sync_copy(stage, degp_hbm.at[c, pl.ds(m * CR, CR), :])


KF = 128
NKF = EPS // KF
KT = EPS - NKF * KF


@functools.partial(
    pl.kernel,
    out_type=jax.ShapeDtypeStruct((NC, N, D), jnp.float32),
    mesh=_mesh(),
    scratch_types=[
        pltpu.VMEM((EPS,), jnp.int32),
        pltpu.VMEM((EPS,), jnp.int32),
        pltpu.VMEM((2, KF), jnp.int32),
        pltpu.VMEM((2, KF, D2), jnp.float32),
        pltpu.VMEM((KT,), jnp.int32),
        pltpu.VMEM((KT, D2), jnp.float32),
        pltpu.VMEM((CR, D2), jnp.float32),
        pltpu.SemaphoreType.DMA((2,)),
        pltpu.VMEM_SHARED((N, D2), jnp.float32),
    ],
    compiler_params=pltpu.CompilerParams(use_tc_tiling_on_sc=False),
)
def _sc_prop(ga_hbm, gb_hbm, src_hbm, dst_hbm, pp_hbm,
             srcbuf, dstbuf, dchbuf, rowsbuf, dchunkt, rowst, zbuf,
             sems, shacc):
    c = lax.axis_index("c")
    s = lax.axis_index("s")
    pltpu.sync_copy(src_hbm.at[s], srcbuf)
    pltpu.sync_copy(dst_hbm.at[s], dstbuf)

    zero16 = jnp.zeros((16,), jnp.float32)

    def zb(i, carry):
        for u in range(D2 // 16):
            zbuf[i, pl.ds(u * 16, 16)] = zero16
        return carry
    lax.fori_loop(0, CR, zb, 0)
    for k in range(CK):
        m = s + k * NS

        @pl.when(m < CH)
        def _():
            pltpu.sync_copy(zbuf, shacc.at[pl.ds(m * CR, CR), :])
    plsc.subcore_barrier()

    def run_edges(gsrc):
        def copyidx(j, slv):
            for u in range(KF // 16):
                dchbuf[slv, pl.ds(u * 16, 16)] = (
                    dstbuf[pl.ds(j * KF + u * 16, 16)])

        def gissue(j, slv):
            pltpu.async_copy(gsrc.at[srcbuf.at[pl.ds(j * KF, KF)]],
                             rowsbuf.at[slv], sems.at[slv])

        def gwait(j, slv):
            pltpu.make_async_copy(gsrc.at[srcbuf.at[pl.ds(j * KF, KF)]],
                                  rowsbuf.at[slv], sems.at[slv]).wait()

        def scat(slv):
            pltpu.sync_copy(rowsbuf.at[slv], shacc.at[dchbuf.at[slv]],
                            add=True)

        copyidx(0, 0)
        gissue(0, 0)

        def body(j, carry):
            slv = lax.rem(j, 2)
            copyidx(j, slv)
            gissue(j, slv)
            slv2 = lax.rem(j + 1, 2)
            gwait(j - 1, slv2)
            scat(slv2)
            return carry
        lax.fori_loop(1, NKF, body, 0)
        gwait(NKF - 1, lax.rem(NKF - 1, 2))
        scat(lax.rem(NKF - 1, 2))

        for u in range(KT // 16):
            dchunkt[pl.ds(u * 16, 16)] = dstbuf[pl.ds(NKF * KF + u * 16, 16)]
        pltpu.async_copy(
            gsrc.at[srcbuf.at[pl.ds(NKF * KF, KT)]], rowst, sems.at[0]).wait()
        pltpu.sync_copy(rowst, shacc.at[dchunkt], add=True)

    @pl.when(c == 0)
    def _():
        run_edges(ga_hbm)

    @pl.when(c == 1)
    def _():
        run_edges(gb_hbm)
    plsc.subcore_barrier()

    for k in range(CK):
        m = s + k * NS

        @pl.when(m < CH)
        def _():
            pltpu.sync_copy(shacc.at[pl.ds(m * CR, CR), :], zbuf)
            pltpu.sync_copy(
                zbuf, pp_hbm.at[c, pl.ds(m * CR, CR), pl.ds(0, D2)])


@functools.partial(
    pl.kernel,
    out_type=[jax.ShapeDtypeStruct((NPAIR, D), jnp.float32),
              jax.ShapeDtypeStruct((NPAIR, D), jnp.float32)],
    mesh=_mesh(),
    scratch_types=[
        pltpu.VMEM((NCP, KP), jnp.int32),
        pltpu.VMEM((NCP, KP), jnp.int32),
        pltpu.VMEM((KP, D), jnp.float32),
        pltpu.VMEM((KP, D), jnp.float32),
        pltpu.VMEM((KP, D), jnp.float32),
        pltpu.VMEM((KP, D), jnp.float32),
        pltpu.SemaphoreType.DMA,
        pltpu.SemaphoreType.DMA,
    ],
)
def _sc_pair(p_hbm, q_hbm, i0_hbm, i1_hbm, ape0_hbm, ape1_hbm,
             i0buf, i1buf, pa, qa, pb, qb, sema, semb):
    c = lax.axis_index("c")
    s = lax.axis_index("s")
    wid = c * NS + s
    pltpu.sync_copy(i0_hbm.at[wid], i0buf)
    pltpu.sync_copy(i1_hbm.at[wid], i1buf)

    base = wid * PPT

    def issue(j, p, q, sem):
        pltpu.async_copy(p_hbm.at[i0buf.at[j]], p, sem)
        pltpu.async_copy(q_hbm.at[i1buf.at[j]], q, sem)

    def waitg(j, p, q, sem):
        pltpu.make_async_copy(p_hbm.at[i0buf.at[j]], p, sem).wait()
        pltpu.make_async_copy(q_hbm.at[i1buf.at[j]], q, sem).wait()

    def outs(j, p, q):
        pltpu.sync_copy(p, ape0_hbm.at[pl.ds(base + j * KP, KP), :])
        pltpu.sync_copy(q, ape1_hbm.at[pl.ds(base + j * KP, KP), :])

    issue(0, pa, qa, sema)
    issue(1, pb, qb, semb)

    def body(t, carry):
        j0 = 2 * t
        j1 = j0 + 1
        waitg(j0, pa, qa, sema)
        outs(j0, pa, qa)

        @pl.when(j0 + 2 < NCP)
        def _():
            issue(j0 + 2, pa, qa, sema)
        waitg(j1, pb, qb, semb)
        outs(j1, pb, qb)

        @pl.when(j1 + 2 < NCP)
        def _():
            issue(j1 + 2, pb, qb, semb)
        return carry
    lax.fori_loop(0, NCP // 2, body, 0)


_RB = 2000
_GRID = N // _RB


def _dinv(degp_ref):
    deg = 1.0 + degp_ref[0, :, 0] + degp_ref[1, :, 0]
    return lax.rsqrt(deg)[:, None]


def _tc_in_body(x_ref, w1_ref, degp_ref, oa_ref, ob_ref):
    g = jnp.dot(x_ref[...], w1_ref[...],
                preferred_element_type=jnp.float32) * _dinv(degp_ref)
    oa_ref[...] = g[:, :D2]
    ob_ref[...] = g[:, D2:]


def _tc_mid_body(pp_ref, ga_ref, gb_ref, degp_ref, b1_ref, w2_ref,
                 oa_ref, ob_ref):
    dinv = _dinv(degp_ref)
    h = jnp.concatenate(
        [pp_ref[0, :, :D2] + ga_ref[...], pp_ref[1, :, :D2] + gb_ref[...]],
        axis=-1)
    h = jnp.maximum(h * dinv + b1_ref[...], 0.0)
    g = jnp.dot(h, w2_ref[...], preferred_element_type=jnp.float32) * dinv
    oa_ref[...] = g[:, :D2]
    ob_ref[...] = g[:, D2:]


def _tc_out_body(pp_ref, ga_ref, gb_ref, degp_ref, b2_ref, wm1_ref,
                 p_ref, q_ref):
    dinv = _dinv(degp_ref)
    h2 = jnp.concatenate(
        [pp_ref[0, :, :D2] + ga_ref[...], pp_ref[1, :, :D2] + gb_ref[...]],
        axis=-1)
    h2 = h2 * dinv + b2_ref[...]
    p_ref[...] = jnp.dot(h2, wm1_ref[:D, :],
                         preferred_element_type=jnp.float32)
    q_ref[...] = jnp.dot(h2, wm1_ref[D:, :],
                         preferred_element_type=jnp.float32)


_PB = 2048


def _tc_head_body(ape0_ref, ape1_ref, bm1_ref, wm2_ref, bm2_ref, o_ref):
    t = jnp.maximum(ape0_ref[...] + ape1_ref[...] + bm1_ref[...], 0.0)
    z = jnp.dot(t, wm2_ref[...], preferred_element_type=jnp.float32)
    o_ref[...] = jax.nn.sigmoid(z + bm2_ref[0, 0])[:, 0]


def _row_spec(shape):
    nd = len(shape)
    if nd == 2:
        return pl.BlockSpec((_RB, shape[1]), lambda i: (i, 0))
    return pl.BlockSpec((shape[0], _RB, shape[2]), lambda i: (0, i, 0))


def _full_spec(shape):
    nd = len(shape)
    return pl.BlockSpec(shape, lambda i: (0,) * nd)


def _tc_in(x, w1, degp):
    half = jax.ShapeDtypeStruct((N, D2), jnp.float32)
    return pl.pallas_call(
        _tc_in_body,
        grid=(_GRID,),
        in_specs=[_row_spec(x.shape), _full_spec(w1.shape),
                  _row_spec(degp.shape)],
        out_specs=[_row_spec((N, D2)), _row_spec((N, D2))],
        out_shape=[half, half],
    )(x, w1, degp)


def _tc_mid(pp, ga, gb, degp, b1, w2):
    half = jax.ShapeDtypeStruct((N, D2), jnp.float32)
    return pl.pallas_call(
        _tc_mid_body,
        grid=(_GRID,),
        in_specs=[_row_spec(pp.shape), _row_spec(ga.shape),
                  _row_spec(gb.shape), _row_spec(degp.shape),
                  _full_spec(b1.shape), _full_spec(w2.shape)],
        out_specs=[_row_spec((N, D2)), _row_spec((N, D2))],
        out_shape=[half, half],
    )(pp, ga, gb, degp, b1, w2)


def _tc_out(pp, ga, gb, degp, b2, wm1):
    full = jax.ShapeDtypeStruct((N, D), jnp.float32)
    return pl.pallas_call(
        _tc_out_body,
        grid=(_GRID,),
        in_specs=[_row_spec(pp.shape), _row_spec(ga.shape),
                  _row_spec(gb.shape), _row_spec(degp.shape),
                  _full_spec(b2.shape), _full_spec(wm1.shape)],
        out_specs=[_row_spec((N, D)), _row_spec((N, D))],
        out_shape=[full, full],
    )(pp, ga, gb, degp, b2, wm1)


def _tc_head(ape0, ape1, bm1, wm2, bm2):
    return pl.pallas_call(
        _tc_head_body,
        grid=(NPAIR // _PB,),
        in_specs=[pl.BlockSpec((_PB, D), lambda i: (i, 0)),
                  pl.BlockSpec((_PB, D), lambda i: (i, 0)),
                  _full_spec(bm1.shape), _full_spec(wm2.shape),
                  _full_spec(bm2.shape)],
        out_specs=pl.BlockSpec((_PB,), lambda i: (i,)),
        out_shape=jax.ShapeDtypeStruct((NPAIR,), jnp.float32),
    )(ape0, ape1, bm1, wm2, bm2)


def kernel(x, edge_index, pair, W1, b1, W2, b2, Wm1, bm1, Wm2, bm2):
    src32 = edge_index[0].astype(jnp.int32)
    dst32 = edge_index[1].astype(jnp.int32)
    dst_deg = dst32.reshape(NW, NCE, KE)
    src_prop = src32.reshape(NS, EPS)
    dst_prop = dst32.reshape(NS, EPS)
    p0 = pair[:, 0].astype(jnp.int32).reshape(NW, NCP, KP)
    p1 = pair[:, 1].astype(jnp.int32).reshape(NW, NCP, KP)

    degp = _sc_degree(dst_deg)
    g1a, g1b = _tc_in(x, W1, degp)
    pp1 = _sc_prop(g1a, g1b, src_prop, dst_prop)
    g2a, g2b = _tc_mid(pp1, g1a, g1b, degp, b1.reshape(1, D), W2)
    pp2 = _sc_prop(g2a, g2b, src_prop, dst_prop)
    P, Q = _tc_out(pp2, g2a, g2b, degp, b2.reshape(1, D), Wm1)
    ape0, ape1 = _sc_pair(P, Q, p0, p1)
    z = _tc_head(ape0, ape1, bm1.reshape(1, D), Wm2, bm2.reshape(1, 1))
    return z.reshape(NPAIR, 1)


# speedup vs baseline: 23.0248x; 1.0258x over previous
import functools

import jax
import jax.numpy as jnp
from jax import lax
from jax.experimental import pallas as pl
from jax.experimental.pallas import tpu as pltpu
from jax.experimental.pallas import tpu_sc as plsc

N = 10000
D = 128
E = 320000
NPAIR = 65536
NC, NS, NW = 2, 16, 32
D2 = D // 2
EPT = E // NW
EPS = E // NS
KE = 80
NCE = EPT // KE
NCP2 = EPS // KE
CR = 400
CH = N // CR
CK = (CH + NS - 1) // NS + 1
KP = 128
PPT = NPAIR // NW
NCP = PPT // KP


def _mesh():
    return plsc.VectorSubcoreMesh(
        core_axis_name="c", subcore_axis_name="s",
        num_cores=NC, num_subcores=NS)


@functools.partial(
    pl.kernel,
    out_type=jax.ShapeDtypeStruct((NC, N, 16), jnp.float32),
    mesh=_mesh(),
    scratch_types=[
        pltpu.VMEM((NCE, KE), jnp.int32),
        pltpu.VMEM((KE,), jnp.int32),
        pltpu.VMEM((KE, 16), jnp.float32),
        pltpu.VMEM((CR, 16), jnp.float32),
        pltpu.VMEM_SHARED((N, 16), jnp.float32),
    ],
    compiler_params=pltpu.CompilerParams(use_tc_tiling_on_sc=False),
)
def _sc_degree(dst_hbm, degp_hbm, idxbuf, idxchunk, ones, stage, shacc):
    c = lax.axis_index("c")
    s = lax.axis_index("s")
    wid = c * NS + s
    pltpu.sync_copy(dst_hbm.at[wid], idxbuf)

    one16 = jnp.ones((16,), jnp.float32)
    zero16 = jnp.zeros((16,), jnp.float32)

    def fill_ones(i, carry):
        ones[i, :] = one16
        return carry
    lax.fori_loop(0, KE, fill_ones, 0)

    def fill_zero(i, carry):
        stage[i, :] = zero16
        return carry
    lax.fori_loop(0, CR, fill_zero, 0)
    for k in range(CK):
        m = s + k * NS

        @pl.when(m < CH)
        def _():
            pltpu.sync_copy(stage, shacc.at[pl.ds(m * CR, CR), :])
    plsc.subcore_barrier()

    def edge_body(j, carry):
        for u in range(KE // 16):
            idxchunk[pl.ds(u * 16, 16)] = idxbuf[j, pl.ds(u * 16, 16)]
        pltpu.sync_copy(ones, shacc.at[idxchunk], add=True)
        return carry
    lax.fori_loop(0, NCE, edge_body, 0)
    plsc.subcore_barrier()

    for k in range(CK):
        m = s + k * NS

        @pl.when(m < CH)
        def _():
            pltpu.sync_copy(shacc.at[pl.ds(m * CR, CR), :], stage)
            pltpu.sync_copy(stage, degp_hbm.at[c, pl.ds(m * CR, CR), :])


KF = 128
NKF = EPS // KF
KT = EPS - NKF * KF


@functools.partial(
    pl.kernel,
    out_type=jax.ShapeDtypeStruct((NC, N, D), jnp.float32),
    mesh=_mesh(),
    scratch_types=[
        pltpu.VMEM((EPS,), jnp.int32),
        pltpu.VMEM((EPS,), jnp.int32),
        pltpu.VMEM((2, KF), jnp.int32),
        pltpu.VMEM((2, KF), jnp.int32),
        pltpu.VMEM((2, KF, D2), jnp.float32),
        pltpu.VMEM((KT,), jnp.int32),
        pltpu.VMEM((KT, D2), jnp.float32),
        pltpu.VMEM((CR, D2), jnp.float32),
        pltpu.SemaphoreType.DMA((2,)),
        pltpu.VMEM_SHARED((N, D2), jnp.float32),
    ],
    compiler_params=pltpu.CompilerParams(use_tc_tiling_on_sc=False),
)
def _sc_prop(ga_hbm, gb_hbm, src_hbm, dst_hbm, pp_hbm,
             srcbuf, dstbuf, dchbuf, schbuf, rowsbuf, dchunkt, rowst, zbuf,
             sems, shacc):
    c = lax.axis_index("c")
    s = lax.axis_index("s")
    pltpu.sync_copy(src_hbm.at[s], srcbuf)
    pltpu.sync_copy(dst_hbm.at[s], dstbuf)

    zero16 = jnp.zeros((16,), jnp.float32)

    def zb(i, carry):
        for u in range(D2 // 16):
            zbuf[i, pl.ds(u * 16, 16)] = zero16
        return carry
    lax.fori_loop(0, CR, zb, 0)
    for k in range(CK):
        m = s + k * NS

        @pl.when(m < CH)
        def _():
            pltpu.sync_copy(zbuf, shacc.at[pl.ds(m * CR, CR), :])
    plsc.subcore_barrier()

    def run_edges(gsrc):
        def copyidx(j, slv):
            for u in range(KF // 16):
                dchbuf[slv, pl.ds(u * 16, 16)] = (
                    dstbuf[pl.ds(j * KF + u * 16, 16)])
                schbuf[slv, pl.ds(u * 16, 16)] = (
                    srcbuf[pl.ds(j * KF + u * 16, 16)] * 2)

        def gissue(slv):
            pltpu.async_copy(gsrc.at[schbuf.at[slv]],
                             rowsbuf.at[slv], sems.at[slv])

        def gwait(slv):
            pltpu.make_async_copy(gsrc.at[schbuf.at[slv]],
                                  rowsbuf.at[slv], sems.at[slv]).wait()

        def scat(slv):
            pltpu.sync_copy(rowsbuf.at[slv], shacc.at[dchbuf.at[slv]],
                            add=True)

        copyidx(0, 0)
        gissue(0)

        def body(j, carry):
            slv = lax.rem(j, 2)
            copyidx(j, slv)
            gissue(slv)
            slv2 = lax.rem(j + 1, 2)
            gwait(slv2)
            scat(slv2)
            return carry
        lax.fori_loop(1, NKF, body, 0)
        gwait((NKF - 1) % 2)
        scat((NKF - 1) % 2)

        for u in range(KT // 16):
            dchunkt[pl.ds(u * 16, 16)] = dstbuf[pl.ds(NKF * KF + u * 16, 16)]
            dchunkt2 = srcbuf[pl.ds(NKF * KF + u * 16, 16)] * 2
            schbuf[0, pl.ds(u * 16, 16)] = dchunkt2
        pltpu.async_copy(
            gsrc.at[schbuf.at[0, pl.ds(0, KT)]], rowst, sems.at[0]).wait()
        pltpu.sync_copy(rowst, shacc.at[dchunkt], add=True)

    @pl.when(c == 0)
    def _():
        run_edges(ga_hbm)

    @pl.when(c == 1)
    def _():
        run_edges(gb_hbm)
    plsc.subcore_barrier()

    for k in range(CK):
        m = s + k * NS

        @pl.when(m < CH)
        def _():
            pltpu.sync_copy(shacc.at[pl.ds(m * CR, CR), :], zbuf)
            pltpu.sync_copy(
                zbuf, pp_hbm.at[c, pl.ds(m * CR, CR), pl.ds(0, D2)])


@functools.partial(
    pl.kernel,
    out_type=[jax.ShapeDtypeStruct((NPAIR, D), jnp.float32),
              jax.ShapeDtypeStruct((NPAIR, D), jnp.float32)],
    mesh=_mesh(),
    scratch_types=[
        pltpu.VMEM((NCP, KP), jnp.int32),
        pltpu.VMEM((NCP, KP), jnp.int32),
        pltpu.VMEM((KP, D), jnp.float32),
        pltpu.VMEM((KP, D), jnp.float32),
        pltpu.VMEM((KP, D), jnp.float32),
        pltpu.VMEM((KP, D), jnp.float32),
        pltpu.SemaphoreType.DMA,
        pltpu.SemaphoreType.DMA,
    ],
)
def _sc_pair(p_hbm, q_hbm, i0_hbm, i1_hbm, ape0_hbm, ape1_hbm,
             i0buf, i1buf, pa, qa, pb, qb, sema, semb):
    c = lax.axis_index("c")
    s = lax.axis_index("s")
    wid = c * NS + s
    pltpu.sync_copy(i0_hbm.at[wid], i0buf)
    pltpu.sync_copy(i1_hbm.at[wid], i1buf)

    base = wid * PPT

    def issue(j, p, q, sem):
        pltpu.async_copy(p_hbm.at[i0buf.at[j]], p, sem)
        pltpu.async_copy(q_hbm.at[i1buf.at[j]], q, sem)

    def waitg(j, p, q, sem):
        pltpu.make_async_copy(p_hbm.at[i0buf.at[j]], p, sem).wait()
        pltpu.make_async_copy(q_hbm.at[i1buf.at[j]], q, sem).wait()

    def outs(j, p, q):
        pltpu.sync_copy(p, ape0_hbm.at[pl.ds(base + j * KP, KP), :])
        pltpu.sync_copy(q, ape1_hbm.at[pl.ds(base + j * KP, KP), :])

    issue(0, pa, qa, sema)
    issue(1, pb, qb, semb)

    def body(t, carry):
        j0 = 2 * t
        j1 = j0 + 1
        waitg(j0, pa, qa, sema)
        outs(j0, pa, qa)

        @pl.when(j0 + 2 < NCP)
        def _():
            issue(j0 + 2, pa, qa, sema)
        waitg(j1, pb, qb, semb)
        outs(j1, pb, qb)

        @pl.when(j1 + 2 < NCP)
        def _():
            issue(j1 + 2, pb, qb, semb)
        return carry
    lax.fori_loop(0, NCP // 2, body, 0)


_RB = 2000
_GRID = N // _RB


def _dinv(degp_ref):
    deg = 1.0 + degp_ref[0, :, 0] + degp_ref[1, :, 0]
    return lax.rsqrt(deg)[:, None]


def _tc_in_body(x_ref, w1_ref, degp_ref, oa_ref, ob_ref):
    g = jnp.dot(x_ref[...], w1_ref[...],
                preferred_element_type=jnp.float32) * _dinv(degp_ref)
    oa_ref[...] = jnp.concatenate([g[:, :D2], g[:, :D2]], axis=-1)
    ob_ref[...] = jnp.concatenate([g[:, D2:], g[:, D2:]], axis=-1)


def _tc_mid_body(pp_ref, ga_ref, gb_ref, degp_ref, b1_ref, w2_ref,
                 oa_ref, ob_ref):
    dinv = _dinv(degp_ref)
    h = jnp.concatenate(
        [pp_ref[0, :, :D2] + ga_ref[:, :D2],
         pp_ref[1, :, :D2] + gb_ref[:, :D2]], axis=-1)
    h = jnp.maximum(h * dinv + b1_ref[...], 0.0)
    g = jnp.dot(h, w2_ref[...], preferred_element_type=jnp.float32) * dinv
    oa_ref[...] = jnp.concatenate([g[:, :D2], g[:, :D2]], axis=-1)
    ob_ref[...] = jnp.concatenate([g[:, D2:], g[:, D2:]], axis=-1)


def _tc_out_body(pp_ref, ga_ref, gb_ref, degp_ref, b2_ref, wm1_ref,
                 p_ref, q_ref):
    dinv = _dinv(degp_ref)
    h2 = jnp.concatenate(
        [pp_ref[0, :, :D2] + ga_ref[:, :D2],
         pp_ref[1, :, :D2] + gb_ref[:, :D2]], axis=-1)
    h2 = h2 * dinv + b2_ref[...]
    p_ref[...] = jnp.dot(h2, wm1_ref[:D, :],
                         preferred_element_type=jnp.float32)
    q_ref[...] = jnp.dot(h2, wm1_ref[D:, :],
                         preferred_element_type=jnp.float32)


_PB = 2048


def _tc_head_body(ape0_ref, ape1_ref, bm1_ref, wm2_ref, bm2_ref, o_ref):
    t = jnp.maximum(ape0_ref[...] + ape1_ref[...] + bm1_ref[...], 0.0)
    z = jnp.dot(t, wm2_ref[...], preferred_element_type=jnp.float32)
    o_ref[...] = jax.nn.sigmoid(z + bm2_ref[0, 0])[:, 0]


def _row_spec(shape):
    nd = len(shape)
    if nd == 2:
        return pl.BlockSpec((_RB, shape[1]), lambda i: (i, 0))
    return pl.BlockSpec((shape[0], _RB, shape[2]), lambda i: (0, i, 0))


def _full_spec(shape):
    nd = len(shape)
    return pl.BlockSpec(shape, lambda i: (0,) * nd)


def _tc_in(x, w1, degp):
    half = jax.ShapeDtypeStruct((N, D), jnp.float32)
    return pl.pallas_call(
        _tc_in_body,
        grid=(_GRID,),
        in_specs=[_row_spec(x.shape), _full_spec(w1.shape),
                  _row_spec(degp.shape)],
        out_specs=[_row_spec((N, D)), _row_spec((N, D))],
        out_shape=[half, half],
    )(x, w1, degp)


def _tc_mid(pp, ga, gb, degp, b1, w2):
    half = jax.ShapeDtypeStruct((N, D), jnp.float32)
    return pl.pallas_call(
        _tc_mid_body,
        grid=(_GRID,),
        in_specs=[_row_spec(pp.shape), _row_spec(ga.shape),
                  _row_spec(gb.shape), _row_spec(degp.shape),
                  _full_spec(b1.shape), _full_spec(w2.shape)],
        out_specs=[_row_spec((N, D)), _row_spec((N, D))],
        out_shape=[half, half],
    )(pp, ga, gb, degp, b1, w2)


def _tc_out(pp, ga, gb, degp, b2, wm1):
    full = jax.ShapeDtypeStruct((N, D), jnp.float32)
    return pl.pallas_call(
        _tc_out_body,
        grid=(_GRID,),
        in_specs=[_row_spec(pp.shape), _row_spec(ga.shape),
                  _row_spec(gb.shape), _row_spec(degp.shape),
                  _full_spec(b2.shape), _full_spec(wm1.shape)],
        out_specs=[_row_spec((N, D)), _row_spec((N, D))],
        out_shape=[full, full],
    )(pp, ga, gb, degp, b2, wm1)


def _tc_head(ape0, ape1, bm1, wm2, bm2):
    return pl.pallas_call(
        _tc_head_body,
        grid=(NPAIR // _PB,),
        in_specs=[pl.BlockSpec((_PB, D), lambda i: (i, 0)),
                  pl.BlockSpec((_PB, D), lambda i: (i, 0)),
                  _full_spec(bm1.shape), _full_spec(wm2.shape),
                  _full_spec(bm2.shape)],
        out_specs=pl.BlockSpec((_PB,), lambda i: (i,)),
        out_shape=jax.ShapeDtypeStruct((NPAIR,), jnp.float32),
    )(ape0, ape1, bm1, wm2, bm2)


def kernel(x, edge_index, pair, W1, b1, W2, b2, Wm1, bm1, Wm2, bm2):
    src32 = edge_index[0].astype(jnp.int32)
    dst32 = edge_index[1].astype(jnp.int32)
    dst_deg = dst32.reshape(NW, NCE, KE)
    src_prop = src32.reshape(NS, EPS)
    dst_prop = dst32.reshape(NS, EPS)
    p0 = pair[:, 0].astype(jnp.int32).reshape(NW, NCP, KP)
    p1 = pair[:, 1].astype(jnp.int32).reshape(NW, NCP, KP)

    degp = _sc_degree(dst_deg)
    g1a, g1b = _tc_in(x, W1, degp)
    pp1 = _sc_prop(g1a.reshape(2 * N, D2), g1b.reshape(2 * N, D2),
                   src_prop, dst_prop)
    g2a, g2b = _tc_mid(pp1, g1a, g1b, degp, b1.reshape(1, D), W2)
    pp2 = _sc_prop(g2a.reshape(2 * N, D2), g2b.reshape(2 * N, D2),
                   src_prop, dst_prop)
    P, Q = _tc_out(pp2, g2a, g2b, degp, b2.reshape(1, D), Wm1)
    ape0, ape1 = _sc_pair(P, Q, p0, p1)
    z = _tc_head(ape0, ape1, bm1.reshape(1, D), Wm2, bm2.reshape(1, 1))
    return z.reshape(NPAIR, 1)


# speedup vs baseline: 23.7392x; 1.0310x over previous
import functools

import jax
import jax.numpy as jnp
from jax import lax
from jax.experimental import pallas as pl
from jax.experimental.pallas import tpu as pltpu
from jax.experimental.pallas import tpu_sc as plsc

N = 10000
D = 128
E = 320000
NPAIR = 65536
NC, NS, NW = 2, 16, 32
D2 = D // 2
EPT = E // NW
EPS = E // NS
KE = 80
NCE = EPT // KE
NCP2 = EPS // KE
CR = 400
CH = N // CR
CK = (CH + NS - 1) // NS + 1
KP = 128
PPT = NPAIR // NW
NCP = PPT // KP


def _mesh():
    return plsc.VectorSubcoreMesh(
        core_axis_name="c", subcore_axis_name="s",
        num_cores=NC, num_subcores=NS)


NR16 = N // 16


@functools.partial(
    pl.kernel,
    out_type=jax.ShapeDtypeStruct((NC, NR16, 16), jnp.float32),
    mesh=_mesh(),
    scratch_types=[
        pltpu.VMEM((EPT,), jnp.int32),
        pltpu.VMEM((640, 16), jnp.float32),
        pltpu.VMEM((5, 128), jnp.int32),
        pltpu.VMEM((25, 16), jnp.float32),
        pltpu.VMEM_SHARED((NR16 + 1, 16), jnp.float32),
    ],
    compiler_params=pltpu.CompilerParams(use_tc_tiling_on_sc=False,
                                         needs_layout_passes=False),
)
def _sc_degree(dst_hbm, degp_hbm, dstbuf, hist, idbuf, stage, shacc):
    c = lax.axis_index("c")
    s = lax.axis_index("s")
    wid = c * NS + s
    pltpu.sync_copy(dst_hbm.at[wid], dstbuf)

    one16 = jnp.ones((16,), jnp.float32)
    zero16 = jnp.zeros((16,), jnp.float32)
    iota16 = lax.iota(jnp.int32, 16)

    def fill_zero(i, carry):
        hist[i, :] = zero16
        return carry
    lax.fori_loop(0, 640, fill_zero, 0)

    @pl.when(s == 0)
    def _():
        pltpu.sync_copy(hist.at[pl.ds(0, NR16 + 1), :], shacc)
    for t in range(5):
        for u in range(8):
            idbuf[t, pl.ds(u * 16, 16)] = jnp.minimum(
                t * 128 + u * 16 + iota16, NR16)
    plsc.subcore_barrier()

    def hbody(i, carry):
        idx = dstbuf[pl.ds(i * 16, 16)]
        plsc.addupdate_scatter(
            hist, [lax.shift_right_logical(idx, 4),
                   lax.bitwise_and(idx, 15)], one16)
        return carry
    lax.fori_loop(0, EPT // 16, hbody, 0)
    plsc.subcore_barrier()

    for t in range(5):
        pltpu.sync_copy(hist.at[pl.ds(t * 128, 128), :],
                        shacc.at[idbuf.at[t]], add=True)
    plsc.subcore_barrier()

    for k in range(CK):
        m = s + k * NS

        @pl.when(m < CH)
        def _():
            pltpu.sync_copy(shacc.at[pl.ds(m * 25, 25), :], stage)
            pltpu.sync_copy(stage, degp_hbm.at[c, pl.ds(m * 25, 25), :])


KF = 128
NKF = EPS // KF
KT = EPS - NKF * KF


@functools.partial(
    pl.kernel,
    out_type=jax.ShapeDtypeStruct((NC, N, D), jnp.float32),
    mesh=_mesh(),
    scratch_types=[
        pltpu.VMEM((EPS,), jnp.int32),
        pltpu.VMEM((EPS,), jnp.int32),
        pltpu.VMEM((2, KF), jnp.int32),
        pltpu.VMEM((2, KF), jnp.int32),
        pltpu.VMEM((2, KF, D2), jnp.float32),
        pltpu.VMEM((KT,), jnp.int32),
        pltpu.VMEM((KT, D2), jnp.float32),
        pltpu.VMEM((CR, D2), jnp.float32),
        pltpu.SemaphoreType.DMA((2,)),
        pltpu.VMEM_SHARED((N, D2), jnp.float32),
    ],
    compiler_params=pltpu.CompilerParams(use_tc_tiling_on_sc=False),
)
def _sc_prop(ga_hbm, gb_hbm, src_hbm, dst_hbm, pp_hbm,
             srcbuf, dstbuf, dchbuf, schbuf, rowsbuf, dchunkt, rowst, zbuf,
             sems, shacc):
    c = lax.axis_index("c")
    s = lax.axis_index("s")
    pltpu.sync_copy(src_hbm.at[s], srcbuf)
    pltpu.sync_copy(dst_hbm.at[s], dstbuf)

    zero16 = jnp.zeros((16,), jnp.float32)

    def zb(i, carry):
        for u in range(D2 // 16):
            zbuf[i, pl.ds(u * 16, 16)] = zero16
        return carry
    lax.fori_loop(0, CR, zb, 0)
    for k in range(CK):
        m = s + k * NS

        @pl.when(m < CH)
        def _():
            pltpu.sync_copy(zbuf, shacc.at[pl.ds(m * CR, CR), :])
    plsc.subcore_barrier()

    def run_edges(gsrc):
        def copyidx(j, slv):
            for u in range(KF // 16):
                dchbuf[slv, pl.ds(u * 16, 16)] = (
                    dstbuf[pl.ds(j * KF + u * 16, 16)])
                schbuf[slv, pl.ds(u * 16, 16)] = (
                    srcbuf[pl.ds(j * KF + u * 16, 16)] * 2)

        def gissue(slv):
            pltpu.async_copy(gsrc.at[schbuf.at[slv]],
                             rowsbuf.at[slv], sems.at[slv])

        def gwait(slv):
            pltpu.make_async_copy(gsrc.at[schbuf.at[slv]],
                                  rowsbuf.at[slv], sems.at[slv]).wait()

        def scat(slv):
            pltpu.sync_copy(rowsbuf.at[slv], shacc.at[dchbuf.at[slv]],
                            add=True)

        copyidx(0, 0)
        gissue(0)

        def body(j, carry):
            slv = lax.rem(j, 2)
            copyidx(j, slv)
            gissue(slv)
            slv2 = lax.rem(j + 1, 2)
            gwait(slv2)
            scat(slv2)
            return carry
        lax.fori_loop(1, NKF, body, 0)
        gwait((NKF - 1) % 2)
        scat((NKF - 1) % 2)

        for u in range(KT // 16):
            dchunkt[pl.ds(u * 16, 16)] = dstbuf[pl.ds(NKF * KF + u * 16, 16)]
            dchunkt2 = srcbuf[pl.ds(NKF * KF + u * 16, 16)] * 2
            schbuf[0, pl.ds(u * 16, 16)] = dchunkt2
        pltpu.async_copy(
            gsrc.at[schbuf.at[0, pl.ds(0, KT)]], rowst, sems.at[0]).wait()
        pltpu.sync_copy(rowst, shacc.at[dchunkt], add=True)

    @pl.when(c == 0)
    def _():
        run_edges(ga_hbm)

    @pl.when(c == 1)
    def _():
        run_edges(gb_hbm)
    plsc.subcore_barrier()

    for k in range(CK):
        m = s + k * NS

        @pl.when(m < CH)
        def _():
            pltpu.sync_copy(shacc.at[pl.ds(m * CR, CR), :], zbuf)
            pltpu.sync_copy(
                zbuf, pp_hbm.at[c, pl.ds(m * CR, CR), pl.ds(0, D2)])


@functools.partial(
    pl.kernel,
    out_type=[jax.ShapeDtypeStruct((NPAIR, D), jnp.float32),
              jax.ShapeDtypeStruct((NPAIR, D), jnp.float32)],
    mesh=_mesh(),
    scratch_types=[
        pltpu.VMEM((NCP, KP), jnp.int32),
        pltpu.VMEM((NCP, KP), jnp.int32),
        pltpu.VMEM((KP, D), jnp.float32),
        pltpu.VMEM((KP, D), jnp.float32),
        pltpu.VMEM((KP, D), jnp.float32),
        pltpu.VMEM((KP, D), jnp.float32),
        pltpu.SemaphoreType.DMA,
        pltpu.SemaphoreType.DMA,
    ],
)
def _sc_pair(p_hbm, q_hbm, i0_hbm, i1_hbm, ape0_hbm, ape1_hbm,
             i0buf, i1buf, pa, qa, pb, qb, sema, semb):
    c = lax.axis_index("c")
    s = lax.axis_index("s")
    wid = c * NS + s
    pltpu.sync_copy(i0_hbm.at[wid], i0buf)
    pltpu.sync_copy(i1_hbm.at[wid], i1buf)

    base = wid * PPT

    def issue(j, p, q, sem):
        pltpu.async_copy(p_hbm.at[i0buf.at[j]], p, sem)
        pltpu.async_copy(q_hbm.at[i1buf.at[j]], q, sem)

    def waitg(j, p, q, sem):
        pltpu.make_async_copy(p_hbm.at[i0buf.at[j]], p, sem).wait()
        pltpu.make_async_copy(q_hbm.at[i1buf.at[j]], q, sem).wait()

    def outs(j, p, q):
        pltpu.sync_copy(p, ape0_hbm.at[pl.ds(base + j * KP, KP), :])
        pltpu.sync_copy(q, ape1_hbm.at[pl.ds(base + j * KP, KP), :])

    issue(0, pa, qa, sema)
    issue(1, pb, qb, semb)

    def body(t, carry):
        j0 = 2 * t
        j1 = j0 + 1
        waitg(j0, pa, qa, sema)
        outs(j0, pa, qa)

        @pl.when(j0 + 2 < NCP)
        def _():
            issue(j0 + 2, pa, qa, sema)
        waitg(j1, pb, qb, semb)
        outs(j1, pb, qb)

        @pl.when(j1 + 2 < NCP)
        def _():
            issue(j1 + 2, pb, qb, semb)
        return carry
    lax.fori_loop(0, NCP // 2, body, 0)


_RB = 2000
_GRID = N // _RB


def _dinv(degp_ref):
    deg = 1.0 + degp_ref[:, 0] + degp_ref[:, 1]
    return lax.rsqrt(deg)[:, None]


def _tc_in_body(x_ref, w1_ref, degp_ref, oa_ref, ob_ref):
    g = jnp.dot(x_ref[...], w1_ref[...],
                preferred_element_type=jnp.float32) * _dinv(degp_ref)
    oa_ref[...] = jnp.concatenate([g[:, :D2], g[:, :D2]], axis=-1)
    ob_ref[...] = jnp.concatenate([g[:, D2:], g[:, D2:]], axis=-1)


def _tc_mid_body(pp_ref, ga_ref, gb_ref, degp_ref, b1_ref, w2_ref,
                 oa_ref, ob_ref):
    dinv = _dinv(degp_ref)
    h = jnp.concatenate(
        [pp_ref[0, :, :D2] + ga_ref[:, :D2],
         pp_ref[1, :, :D2] + gb_ref[:, :D2]], axis=-1)
    h = jnp.maximum(h * dinv + b1_ref[...], 0.0)
    g = jnp.dot(h, w2_ref[...], preferred_element_type=jnp.float32) * dinv
    oa_ref[...] = jnp.concatenate([g[:, :D2], g[:, :D2]], axis=-1)
    ob_ref[...] = jnp.concatenate([g[:, D2:], g[:, D2:]], axis=-1)


def _tc_out_body(pp_ref, ga_ref, gb_ref, degp_ref, b2_ref, wm1_ref,
                 p_ref, q_ref):
    dinv = _dinv(degp_ref)
    h2 = jnp.concatenate(
        [pp_ref[0, :, :D2] + ga_ref[:, :D2],
         pp_ref[1, :, :D2] + gb_ref[:, :D2]], axis=-1)
    h2 = h2 * dinv + b2_ref[...]
    p_ref[...] = jnp.dot(h2, wm1_ref[:D, :],
                         preferred_element_type=jnp.float32)
    q_ref[...] = jnp.dot(h2, wm1_ref[D:, :],
                         preferred_element_type=jnp.float32)


_PB = 2048


def _tc_head_body(ape0_ref, ape1_ref, bm1_ref, wm2_ref, bm2_ref, o_ref):
    t = jnp.maximum(ape0_ref[...] + ape1_ref[...] + bm1_ref[...], 0.0)
    z = jnp.dot(t, wm2_ref[...], preferred_element_type=jnp.float32)
    o_ref[...] = jax.nn.sigmoid(z + bm2_ref[0, 0])[:, 0]


def _row_spec(shape):
    nd = len(shape)
    if nd == 2:
        if shape == (N, NC):
            return pl.BlockSpec((_RB, NC), lambda i: (i, 0))
        return pl.BlockSpec((_RB, shape[1]), lambda i: (i, 0))
    return pl.BlockSpec((shape[0], _RB, shape[2]), lambda i: (0, i, 0))


def _full_spec(shape):
    nd = len(shape)
    return pl.BlockSpec(shape, lambda i: (0,) * nd)


def _tc_in(x, w1, degp):
    half = jax.ShapeDtypeStruct((N, D), jnp.float32)
    return pl.pallas_call(
        _tc_in_body,
        grid=(_GRID,),
        in_specs=[_row_spec(x.shape), _full_spec(w1.shape),
                  _row_spec(degp.shape)],
        out_specs=[_row_spec((N, D)), _row_spec((N, D))],
        out_shape=[half, half],
    )(x, w1, degp)


def _tc_mid(pp, ga, gb, degp, b1, w2):
    half = jax.ShapeDtypeStruct((N, D), jnp.float32)
    return pl.pallas_call(
        _tc_mid_body,
        grid=(_GRID,),
        in_specs=[_row_spec(pp.shape), _row_spec(ga.shape),
                  _row_spec(gb.shape), _row_spec(degp.shape),
                  _full_spec(b1.shape), _full_spec(w2.shape)],
        out_specs=[_row_spec((N, D)), _row_spec((N, D))],
        out_shape=[half, half],
    )(pp, ga, gb, degp, b1, w2)


def _tc_out(pp, ga, gb, degp, b2, wm1):
    full = jax.ShapeDtypeStruct((N, D), jnp.float32)
    return pl.pallas_call(
        _tc_out_body,
        grid=(_GRID,),
        in_specs=[_row_spec(pp.shape), _row_spec(ga.shape),
                  _row_spec(gb.shape), _row_spec(degp.shape),
                  _full_spec(b2.shape), _full_spec(wm1.shape)],
        out_specs=[_row_spec((N, D)), _row_spec((N, D))],
        out_shape=[full, full],
    )(pp, ga, gb, degp, b2, wm1)


def _tc_head(ape0, ape1, bm1, wm2, bm2):
    return pl.pallas_call(
        _tc_head_body,
        grid=(NPAIR // _PB,),
        in_specs=[pl.BlockSpec((_PB, D), lambda i: (i, 0)),
                  pl.BlockSpec((_PB, D), lambda i: (i, 0)),
                  _full_spec(bm1.shape), _full_spec(wm2.shape),
                  _full_spec(bm2.shape)],
        out_specs=pl.BlockSpec((_PB,), lambda i: (i,)),
        out_shape=jax.ShapeDtypeStruct((NPAIR,), jnp.float32),
    )(ape0, ape1, bm1, wm2, bm2)


def kernel(x, edge_index, pair, W1, b1, W2, b2, Wm1, bm1, Wm2, bm2):
    src32 = edge_index[0].astype(jnp.int32)
    dst32 = edge_index[1].astype(jnp.int32)
    dst_deg = dst32.reshape(NW, EPT)
    src_prop = src32.reshape(NS, EPS)
    dst_prop = dst32.reshape(NS, EPS)
    p0 = pair[:, 0].astype(jnp.int32).reshape(NW, NCP, KP)
    p1 = pair[:, 1].astype(jnp.int32).reshape(NW, NCP, KP)

    degp = _sc_degree(dst_deg).reshape(NC, N).T
    g1a, g1b = _tc_in(x, W1, degp)
    pp1 = _sc_prop(g1a.reshape(2 * N, D2), g1b.reshape(2 * N, D2),
                   src_prop, dst_prop)
    g2a, g2b = _tc_mid(pp1, g1a, g1b, degp, b1.reshape(1, D), W2)
    pp2 = _sc_prop(g2a.reshape(2 * N, D2), g2b.reshape(2 * N, D2),
                   src_prop, dst_prop)
    P, Q = _tc_out(pp2, g2a, g2b, degp, b2.reshape(1, D), Wm1)
    ape0, ape1 = _sc_pair(P, Q, p0, p1)
    z = _tc_head(ape0, ape1, bm1.reshape(1, D), Wm2, bm2.reshape(1, 1))
    return z.reshape(NPAIR, 1)
